# baseline (speedup 1.0000x reference)
import functools

import jax
import jax.numpy as jnp
from jax import lax
from jax.experimental import pallas as pl
from jax.experimental.pallas import tpu as pltpu
from jax.experimental.pallas import tpu_sc as plsc

DIM = 128
N_NODES = 10000
N_EDGES = 160000
N_TRIPLETS = 320000

NC = 2
NS = 16
NW = NC * NS

F32 = jnp.float32


def _silu(x):
    return x * jax.nn.sigmoid(x)


def _dot(a, b):
    return jnp.dot(a, b, preferred_element_type=F32)


def _row_spec(blk):
    return pl.BlockSpec((blk, DIM), lambda b: (b, 0))


def _w_spec(shape):
    return pl.BlockSpec(shape, lambda b: tuple(0 for _ in shape))


def _node1_body(h_ref, w_ref, b_ref, o_ref):
    o_ref[:] = _silu(_dot(h_ref[:], w_ref[:]) + b_ref[:])


def _tc_node1(h, w, b):
    n, blk = h.shape[0], 2000
    return pl.pallas_call(
        _node1_body,
        grid=(n // blk,),
        in_specs=[_row_spec(blk), _w_spec((DIM, DIM)), _w_spec((1, DIM))],
        out_specs=_row_spec(blk),
        out_shape=jax.ShapeDtypeStruct((n, DIM), F32),
    )(h, w, b)


def _edge1_body(hi, hj, rbf, wa1, wb1, wc1, b1, wa2, wb2, wc2, b2, l1,
                t1_o, mji1_o):
    x = rbf[:]
    mkj = _silu(_dot(hi[:], wa1[:]) + _dot(hj[:], wb1[:]) + _dot(x, wc1[:])
                + b1[:])
    t1_o[:] = mkj * _dot(x, l1[:])
    mji1_o[:] = _silu(_dot(hi[:], wa2[:]) + _dot(hj[:], wb2[:])
                      + _dot(x, wc2[:]) + b2[:])


def _tc_edge1(hi, hj, rbf, wa1, wb1, wc1, b1, wa2, wb2, wc2, b2, l1):
    n, blk = rbf.shape[0], 640
    ws = [_w_spec((DIM, DIM))] * 3 + [_w_spec((1, DIM))]
    return pl.pallas_call(
        _edge1_body,
        grid=(n // blk,),
        in_specs=[_row_spec(blk)] * 3 + ws + ws + [_w_spec((DIM, DIM))],
        out_specs=[_row_spec(blk)] * 2,
        out_shape=[jax.ShapeDtypeStruct((n, DIM), F32)] * 2,
    )(hi, hj, rbf, wa1, wb1, wc1, b1, wa2, wb2, wc2, b2, l1)


def _trip_body(sbf, g, w1, b1, w2, b2, v_o):
    x = _silu(_dot(sbf[:], w1[:]) + b1[:])
    x = _silu(_dot(x, w2[:]) + b2[:])
    v_o[:] = x * g[:]


def _tc_trip(sbf, g, w1, b1, w2, b2):
    n, blk = sbf.shape[0], 640
    return pl.pallas_call(
        _trip_body,
        grid=(n // blk,),
        in_specs=[_row_spec(blk)] * 2 + [_w_spec((DIM, DIM)), _w_spec((1, DIM)),
                                         _w_spec((DIM, DIM)), _w_spec((1, DIM))],
        out_specs=_row_spec(blk),
        out_shape=jax.ShapeDtypeStruct((n, DIM), F32),
    )(sbf, g, w1, b1, w2, b2)


def _edge2_body(mji1, agg1, rbf, wjj, bjj, l2, wji2, bji2, l3,
                t2_o, mji2_o, r3_o):
    m2 = mji1[:] + agg1[:]
    x = rbf[:]
    t2_o[:] = _silu(_dot(m2, wjj[:]) + bjj[:]) * _dot(x, l2[:])
    mji2_o[:] = _silu(_dot(m2, wji2[:]) + bji2[:])
    r3_o[:] = _dot(x, l3[:])


def _tc_edge2(mji1, agg1, rbf, wjj, bjj, l2, wji2, bji2, l3):
    n, blk = rbf.shape[0], 640
    return pl.pallas_call(
        _edge2_body,
        grid=(n // blk,),
        in_specs=[_row_spec(blk)] * 3
        + [_w_spec((DIM, DIM)), _w_spec((1, DIM)), _w_spec((DIM, DIM)),
           _w_spec((DIM, DIM)), _w_spec((1, DIM)), _w_spec((DIM, DIM))],
        out_specs=[_row_spec(blk)] * 3,
        out_shape=[jax.ShapeDtypeStruct((n, DIM), F32)] * 3,
    )(mji1, agg1, rbf, wjj, bjj, l2, wji2, bji2, l3)


def _edge3_body(mji2, agg2, r3, m3_o):
    m3_o[:] = r3[:] * (mji2[:] + agg2[:])


def _tc_edge3(mji2, agg2, r3):
    n, blk = mji2.shape[0], 2000
    return pl.pallas_call(
        _edge3_body,
        grid=(n // blk,),
        in_specs=[_row_spec(blk)] * 3,
        out_specs=_row_spec(blk),
        out_shape=jax.ShapeDtypeStruct((n, DIM), F32),
    )(mji2, agg2, r3)


def _node2_body(pa, pb, h,
                r1w1, r1b1, r1w2, r1b2,
                hw, hb,
                r2w1, r2b1, r2w2, r2b2,
                r3w1, r3b1, r3w2, r3b2,
                yw1, yb1, yw2, yb2, yw3, yb3,
                wy, by,
                h_o, y_o):
    def res(x, w1, b1, w2, b2):
        z = _silu(_dot(x, w1[:]) + b1[:])
        z = _silu(_dot(z, w2[:]) + b2[:])
        return z + x

    t = pa[:] + pb[:]
    t = res(t, r1w1, r1b1, r1w2, r1b2)
    t = _silu(_dot(t, hw[:]) + hb[:]) + h[:]
    t = res(t, r2w1, r2b1, r2w2, r2b2)
    t = res(t, r3w1, r3b1, r3w2, r3b2)
    h_o[:] = t
    z = _silu(_dot(t, yw1[:]) + yb1[:])
    z = _silu(_dot(z, yw2[:]) + yb2[:])
    z = _silu(_dot(z, yw3[:]) + yb3[:])
    y_o[:] = _dot(z, wy[:]) + by[:]


def _tc_node2(pa, pb, h, weights):
    n, blk = h.shape[0], 2000
    wspecs = []
    for w in weights:
        wspecs.append(_w_spec(w.shape))
    return pl.pallas_call(
        _node2_body,
        grid=(n // blk,),
        in_specs=[_row_spec(blk)] * 3 + wspecs,
        out_specs=[_row_spec(blk), pl.BlockSpec((blk, 1), lambda b: (b, 0))],
        out_shape=[jax.ShapeDtypeStruct((n, DIM), F32),
                   jax.ShapeDtypeStruct((n, 1), F32)],
    )(pa, pb, h, *weights)


_SC_MESH = dict(core_axis_name="c", subcore_axis_name="s",
                num_cores=NC, num_subcores=NS)
_WIN = 128


def _sc_gather(table, idx):
    b = idx.shape[0]
    per_w = b // NW
    n_full, tail = divmod(per_w, _WIN)
    mesh = plsc.VectorSubcoreMesh(**_SC_MESH)

    scratch = [
        pltpu.VMEM((_WIN,), jnp.int32),
        pltpu.VMEM((_WIN, DIM), F32),
        pltpu.SemaphoreType.DMA,
    ]
    if tail:
        scratch += [pltpu.VMEM((tail,), jnp.int32), pltpu.VMEM((tail, DIM), F32)]

    @functools.partial(
        pl.kernel, mesh=mesh,
        out_type=jax.ShapeDtypeStruct((b, DIM), F32),
        scratch_types=scratch,
    )
    def k(table_hbm, idx_hbm, out_hbm, idx_v, rows_v, sem, *tail_bufs):
        wid = lax.axis_index("s") * NC + lax.axis_index("c")
        base = wid * per_w

        def win(off, w, iv, rv):
            pltpu.sync_copy(idx_hbm.at[pl.ds(off, w)], iv)
            pltpu.async_copy(table_hbm.at[iv], rv, sem).wait()
            pltpu.sync_copy(rv, out_hbm.at[pl.ds(off, w)])

        def body(iw, carry):
            win(base + iw * _WIN, _WIN, idx_v, rows_v)
            return carry

        lax.fori_loop(0, n_full, body, 0)
        if tail:
            win(base + n_full * _WIN, tail, tail_bufs[0], tail_bufs[1])

    return k(table, idx)


_EC_C = 13440
_EC_DUMMY = 256
_EC_ROWS = _EC_C + _EC_DUMMY
_EC_CHUNKS = [(b, min(_EC_C, N_EDGES - b)) for b in range(0, N_EDGES, _EC_C)]


def _sc_segsum_edges(v, dst, zeros):
    t = v.shape[0]
    per_sc = len(_EC_CHUNKS) // NC
    per_tile = t // NS
    n_full, tail = divmod(per_tile, _WIN)
    zrows_tile = _EC_ROWS // NS
    mesh = plsc.VectorSubcoreMesh(**_SC_MESH)

    scratch = [
        pltpu.VMEM((_WIN,), jnp.int32),
        pltpu.VMEM((_WIN,), jnp.int32),
        pltpu.VMEM((_WIN, DIM), F32),
        pltpu.VMEM_SHARED((_EC_ROWS, DIM), F32),
    ]
    if tail:
        scratch += [pltpu.VMEM((tail,), jnp.int32),
                    pltpu.VMEM((tail,), jnp.int32),
                    pltpu.VMEM((tail, DIM), F32)]

    @functools.partial(
        pl.kernel, mesh=mesh,
        out_type=jax.ShapeDtypeStruct((N_EDGES, DIM), F32),
        scratch_types=scratch,
    )
    def k(v_hbm, dst_hbm, z_hbm, out_hbm, idx_v, loc_v, val_v, acc, *tail_bufs):
        cid = lax.axis_index("c")
        sid = lax.axis_index("s")
        tbase = sid * per_tile

        def win(off, w, nv, iv, lv, vv, cbase):
            pltpu.sync_copy(dst_hbm.at[pl.ds(off, w)], iv)
            pltpu.sync_copy(v_hbm.at[pl.ds(off, w)], vv)
            for kk in range(nv):
                dv = iv[pl.ds(kk * 16, 16)]
                loc = dv - cbase
                ok = (loc >= 0) & (loc < _EC_C)
                dummy = _EC_C + (dv & (_EC_DUMMY - 1))
                lv[pl.ds(kk * 16, 16)] = jnp.where(ok, loc, dummy)
            pltpu.sync_copy(vv, acc.at[lv], add=True)

        def writeout(cbase, csize):
            orows_tile = csize // NS
            pltpu.sync_copy(
                acc.at[pl.ds(sid * orows_tile, orows_tile)],
                out_hbm.at[pl.ds(cbase + sid * orows_tile, orows_tile)])

        for p in range(per_sc):
            chunk = cid * per_sc + p
            cbase = chunk * _EC_C
            pltpu.sync_copy(
                z_hbm.at[pl.ds(sid * zrows_tile, zrows_tile)],
                acc.at[pl.ds(sid * zrows_tile, zrows_tile)])
            plsc.subcore_barrier()

            def body(iw, carry):
                win(tbase + iw * _WIN, _WIN, _WIN // 16,
                    idx_v, loc_v, val_v, cbase)
                return carry

            lax.fori_loop(0, n_full, body, 0)
            if tail:
                win(tbase + n_full * _WIN, tail, tail // 16,
                    tail_bufs[0], tail_bufs[1], tail_bufs[2], cbase)
            plsc.subcore_barrier()
            if p < per_sc - 1:
                writeout(cbase, _EC_C)
            else:
                last0 = _EC_CHUNKS[per_sc - 1]
                last1 = _EC_CHUNKS[NC * per_sc - 1]

                @pl.when(cid == 0)
                def _w0():
                    writeout(last0[0], last0[1])

                @pl.when(cid == 1)
                def _w1():
                    writeout(last1[0], last1[1])
            plsc.subcore_barrier()

    return k(v, dst, zeros)


def _sc_segsum_nodes(v, dst, zeros):
    e = v.shape[0]
    per_sc = e // NC
    per_tile = per_sc // NS
    n_full, tail = divmod(per_tile, _WIN)
    nrows = 10240
    zrows_tile = nrows // NS
    mesh = plsc.VectorSubcoreMesh(**_SC_MESH)

    scratch = [
        pltpu.VMEM((_WIN,), jnp.int32),
        pltpu.VMEM((_WIN, DIM), F32),
        pltpu.VMEM_SHARED((nrows, DIM), F32),
    ]
    if tail:
        scratch += [pltpu.VMEM((tail,), jnp.int32), pltpu.VMEM((tail, DIM), F32)]

    @functools.partial(
        pl.kernel, mesh=mesh,
        out_type=jax.ShapeDtypeStruct((NC * nrows, DIM), F32),
        scratch_types=scratch,
    )
    def k(v_hbm, dst_hbm, z_hbm, out_hbm, idx_v, val_v, acc, *tail_bufs):
        cid = lax.axis_index("c")
        sid = lax.axis_index("s")
        tbase = cid * per_sc + sid * per_tile

        pltpu.sync_copy(z_hbm.at[pl.ds(sid * zrows_tile, zrows_tile)],
                        acc.at[pl.ds(sid * zrows_tile, zrows_tile)])
        plsc.subcore_barrier()

        def win(off, w, iv, vv):
            pltpu.sync_copy(dst_hbm.at[pl.ds(off, w)], iv)
            pltpu.sync_copy(v_hbm.at[pl.ds(off, w)], vv)
            pltpu.sync_copy(vv, acc.at[iv], add=True)

        def body(iw, carry):
            win(tbase + iw * _WIN, _WIN, idx_v, val_v)
            return carry

        lax.fori_loop(0, n_full, body, 0)
        if tail:
            win(tbase + n_full * _WIN, tail, tail_bufs[0], tail_bufs[1])
        plsc.subcore_barrier()
        pltpu.sync_copy(
            acc.at[pl.ds(sid * zrows_tile, zrows_tile)],
            out_hbm.at[pl.ds(cid * nrows + sid * zrows_tile, zrows_tile)])

    return k(v, dst, zeros)


def kernel(h, rbf, sbf1, sbf2, idx_kj, idx_ji_1, idx_jj, idx_ji_2,
           edge_index, params):
    p = params
    i32 = jnp.int32
    j = edge_index[0].astype(i32)
    i = edge_index[1].astype(i32)
    idx_kj = idx_kj.astype(i32)
    idx_ji_1 = idx_ji_1.astype(i32)
    idx_jj = idx_jj.astype(i32)
    idx_ji_2 = idx_ji_2.astype(i32)

    def wb(layer):
        w, b = layer
        return w, b.reshape(1, DIM)

    wh, bh = wb(p['h_mlp'][0])
    wkj, bkj = wb(p['mlp_kj'][0])
    wj1, bj1 = wb(p['mlp_ji_1'][0])
    wjj, bjj = wb(p['mlp_jj'][0])
    wj2, bj2 = wb(p['mlp_ji_2'][0])
    s1w1, s1b1 = wb(p['mlp_sbf1'][0])
    s1w2, s1b2 = wb(p['mlp_sbf1'][1])
    s2w1, s2b1 = wb(p['mlp_sbf2'][0])
    s2w2, s2b2 = wb(p['mlp_sbf2'][1])

    zeros = jnp.zeros((_EC_ROWS, DIM), F32)

    hh = _tc_node1(h, wh, bh)
    hh_i = _sc_gather(hh, i)
    hh_j = _sc_gather(hh, j)

    t1, mji1 = _tc_edge1(
        hh_i, hh_j, rbf,
        wkj[:DIM], wkj[DIM:2 * DIM], wkj[2 * DIM:], bkj,
        wj1[:DIM], wj1[DIM:2 * DIM], wj1[2 * DIM:], bj1,
        p['lin_rbf1'])

    g1 = _sc_gather(t1, idx_kj)
    v1 = _tc_trip(sbf1, g1, s1w1, s1b1, s1w2, s1b2)
    agg1 = _sc_segsum_edges(v1, idx_ji_1, zeros)

    t2, mji2, r3 = _tc_edge2(mji1, agg1, rbf, wjj, bjj, p['lin_rbf2'],
                             wj2, bj2, p['lin_rbf_out'])

    g2 = _sc_gather(t2, idx_jj)
    v2 = _tc_trip(sbf2, g2, s2w1, s2b1, s2w2, s2b2)
    agg2 = _sc_segsum_edges(v2, idx_ji_2, zeros)

    m3 = _tc_edge3(mji2, agg2, r3)
    hparts = _sc_segsum_nodes(m3, i, zeros)
    pa = hparts[:N_NODES]
    pb = hparts[10240:10240 + N_NODES]

    weights = []
    for (w1, b1), (w2, b2) in [(p['res1'][0], p['res1'][1])]:
        weights += [w1, b1.reshape(1, DIM), w2, b2.reshape(1, DIM)]
    weights += [wh, bh]
    for key in ('res2', 'res3'):
        (w1, b1), (w2, b2) = p[key]
        weights += [w1, b1.reshape(1, DIM), w2, b2.reshape(1, DIM)]
    for w, b in p['y_mlp']:
        weights += [w, b.reshape(1, DIM)]
    wy, by = p['y_W']
    weights += [wy, by.reshape(1, 1)]

    h_out, y = _tc_node2(pa, pb, h, weights)
    return (h_out, y)


# speedup vs baseline: 1.3318x; 1.3318x over previous
import functools

import jax
import jax.numpy as jnp
from jax import lax
from jax.experimental import pallas as pl
from jax.experimental.pallas import tpu as pltpu
from jax.experimental.pallas import tpu_sc as plsc

DIM = 128
N_NODES = 10000
N_EDGES = 160000
N_TRIPLETS = 320000

NC = 2
NS = 16
NW = NC * NS

F32 = jnp.float32


def _silu(x):
    return x * jax.nn.sigmoid(x)


def _dot(a, b):
    return jnp.dot(a, b, preferred_element_type=F32)


def _row_spec(blk):
    return pl.BlockSpec((blk, DIM), lambda b: (b, 0))


def _w_spec(shape):
    return pl.BlockSpec(shape, lambda b: tuple(0 for _ in shape))


def _node1_body(h_ref, w_ref, b_ref, o_ref):
    o_ref[:] = _silu(_dot(h_ref[:], w_ref[:]) + b_ref[:])


def _tc_node1(h, w, b):
    n, blk = h.shape[0], 2000
    return pl.pallas_call(
        _node1_body,
        grid=(n // blk,),
        in_specs=[_row_spec(blk), _w_spec((DIM, DIM)), _w_spec((1, DIM))],
        out_specs=_row_spec(blk),
        out_shape=jax.ShapeDtypeStruct((n, DIM), F32),
    )(h, w, b)


def _edge1_body(hi, hj, rbf, wa1, wb1, wc1, b1, wa2, wb2, wc2, b2, l1,
                t1_o, mji1_o):
    x = rbf[:]
    mkj = _silu(_dot(hi[:], wa1[:]) + _dot(hj[:], wb1[:]) + _dot(x, wc1[:])
                + b1[:])
    t1_o[:] = mkj * _dot(x, l1[:])
    mji1_o[:] = _silu(_dot(hi[:], wa2[:]) + _dot(hj[:], wb2[:])
                      + _dot(x, wc2[:]) + b2[:])


def _tc_edge1(hi, hj, rbf, wa1, wb1, wc1, b1, wa2, wb2, wc2, b2, l1):
    n, blk = rbf.shape[0], 640
    ws = [_w_spec((DIM, DIM))] * 3 + [_w_spec((1, DIM))]
    return pl.pallas_call(
        _edge1_body,
        grid=(n // blk,),
        in_specs=[_row_spec(blk)] * 3 + ws + ws + [_w_spec((DIM, DIM))],
        out_specs=[_row_spec(blk)] * 2,
        out_shape=[jax.ShapeDtypeStruct((n, DIM), F32)] * 2,
    )(hi, hj, rbf, wa1, wb1, wc1, b1, wa2, wb2, wc2, b2, l1)


def _trip_body(sbf, g, w1, b1, w2, b2, v_o):
    x = _silu(_dot(sbf[:], w1[:]) + b1[:])
    x = _silu(_dot(x, w2[:]) + b2[:])
    v_o[:] = x * g[:]


def _tc_trip(sbf, g, w1, b1, w2, b2):
    n, blk = sbf.shape[0], 640
    return pl.pallas_call(
        _trip_body,
        grid=(n // blk,),
        in_specs=[_row_spec(blk)] * 2 + [_w_spec((DIM, DIM)), _w_spec((1, DIM)),
                                         _w_spec((DIM, DIM)), _w_spec((1, DIM))],
        out_specs=_row_spec(blk),
        out_shape=jax.ShapeDtypeStruct((n, DIM), F32),
    )(sbf, g, w1, b1, w2, b2)


def _edge2_body(mji1, agg1, rbf, wjj, bjj, l2, wji2, bji2, l3,
                t2_o, mji2_o, r3_o):
    m2 = mji1[:] + agg1[:]
    x = rbf[:]
    t2_o[:] = _silu(_dot(m2, wjj[:]) + bjj[:]) * _dot(x, l2[:])
    mji2_o[:] = _silu(_dot(m2, wji2[:]) + bji2[:])
    r3_o[:] = _dot(x, l3[:])


def _tc_edge2(mji1, agg1, rbf, wjj, bjj, l2, wji2, bji2, l3):
    n, blk = rbf.shape[0], 640
    return pl.pallas_call(
        _edge2_body,
        grid=(n // blk,),
        in_specs=[_row_spec(blk)] * 3
        + [_w_spec((DIM, DIM)), _w_spec((1, DIM)), _w_spec((DIM, DIM)),
           _w_spec((DIM, DIM)), _w_spec((1, DIM)), _w_spec((DIM, DIM))],
        out_specs=[_row_spec(blk)] * 3,
        out_shape=[jax.ShapeDtypeStruct((n, DIM), F32)] * 3,
    )(mji1, agg1, rbf, wjj, bjj, l2, wji2, bji2, l3)


def _edge3_body(mji2, agg2, r3, m3_o):
    m3_o[:] = r3[:] * (mji2[:] + agg2[:])


def _tc_edge3(mji2, agg2, r3):
    n, blk = mji2.shape[0], 2000
    return pl.pallas_call(
        _edge3_body,
        grid=(n // blk,),
        in_specs=[_row_spec(blk)] * 3,
        out_specs=_row_spec(blk),
        out_shape=jax.ShapeDtypeStruct((n, DIM), F32),
    )(mji2, agg2, r3)


def _node2_body(pa, pb, h,
                r1w1, r1b1, r1w2, r1b2,
                hw, hb,
                r2w1, r2b1, r2w2, r2b2,
                r3w1, r3b1, r3w2, r3b2,
                yw1, yb1, yw2, yb2, yw3, yb3,
                wy, by,
                h_o, y_o):
    def res(x, w1, b1, w2, b2):
        z = _silu(_dot(x, w1[:]) + b1[:])
        z = _silu(_dot(z, w2[:]) + b2[:])
        return z + x

    t = pa[:] + pb[:]
    t = res(t, r1w1, r1b1, r1w2, r1b2)
    t = _silu(_dot(t, hw[:]) + hb[:]) + h[:]
    t = res(t, r2w1, r2b1, r2w2, r2b2)
    t = res(t, r3w1, r3b1, r3w2, r3b2)
    h_o[:] = t
    z = _silu(_dot(t, yw1[:]) + yb1[:])
    z = _silu(_dot(z, yw2[:]) + yb2[:])
    z = _silu(_dot(z, yw3[:]) + yb3[:])
    y_o[:] = _dot(z, wy[:]) + by[:]


def _tc_node2(pa, pb, h, weights):
    n, blk = h.shape[0], 2000
    wspecs = []
    for w in weights:
        wspecs.append(_w_spec(w.shape))
    return pl.pallas_call(
        _node2_body,
        grid=(n // blk,),
        in_specs=[_row_spec(blk)] * 3 + wspecs,
        out_specs=[_row_spec(blk), pl.BlockSpec((blk, 1), lambda b: (b, 0))],
        out_shape=[jax.ShapeDtypeStruct((n, DIM), F32),
                   jax.ShapeDtypeStruct((n, 1), F32)],
    )(pa, pb, h, *weights)


_SC_MESH = dict(core_axis_name="c", subcore_axis_name="s",
                num_cores=NC, num_subcores=NS)
_WIN = 128


def _sc_gather(table, idx):
    b = idx.shape[0]
    per_w = b // NW
    n_full, tail = divmod(per_w, _WIN)
    mesh = plsc.VectorSubcoreMesh(**_SC_MESH)

    scratch = [
        pltpu.VMEM((_WIN,), jnp.int32),
        pltpu.VMEM((_WIN, DIM), F32),
        pltpu.SemaphoreType.DMA,
    ]
    if tail:
        scratch += [pltpu.VMEM((tail,), jnp.int32), pltpu.VMEM((tail, DIM), F32)]

    @functools.partial(
        pl.kernel, mesh=mesh,
        out_type=jax.ShapeDtypeStruct((b, DIM), F32),
        scratch_types=scratch,
    )
    def k(table_hbm, idx_hbm, out_hbm, idx_v, rows_v, sem, *tail_bufs):
        wid = lax.axis_index("s") * NC + lax.axis_index("c")
        base = wid * per_w

        def win(off, w, iv, rv):
            pltpu.sync_copy(idx_hbm.at[pl.ds(off, w)], iv)
            pltpu.async_copy(table_hbm.at[iv], rv, sem).wait()
            pltpu.sync_copy(rv, out_hbm.at[pl.ds(off, w)])

        def body(iw, carry):
            win(base + iw * _WIN, _WIN, idx_v, rows_v)
            return carry

        lax.fori_loop(0, n_full, body, 0)
        if tail:
            win(base + n_full * _WIN, tail, tail_bufs[0], tail_bufs[1])

    return k(table, idx)


_EC_C = 10240
_EC_DUMMY = 256
_EC_ROWS = _EC_C + _EC_DUMMY
_EC_CHUNKS = [(b, min(_EC_C, N_EDGES - b)) for b in range(0, N_EDGES, _EC_C)]


def _sc_segsum_edges(v, dst, zeros):
    t = v.shape[0]
    per_sc = len(_EC_CHUNKS) // NC
    per_tile = t // NS
    n_full, tail = divmod(per_tile, _WIN)
    zrows_tile = _EC_ROWS // NS
    mesh = plsc.VectorSubcoreMesh(**_SC_MESH)

    n_half = n_full // 2
    assert n_full == 2 * n_half, "window count must be even for 2-deep ring"

    scratch = [
        [pltpu.VMEM((_WIN,), jnp.int32)] * 2,
        [pltpu.VMEM((_WIN,), jnp.int32)] * 2,
        [pltpu.VMEM((_WIN, DIM), F32)] * 2,
        [pltpu.SemaphoreType.DMA] * 2,
        [pltpu.SemaphoreType.DMA] * 2,
        pltpu.VMEM_SHARED((_EC_ROWS, DIM), F32),
    ]
    if tail:
        scratch += [pltpu.VMEM((tail,), jnp.int32),
                    pltpu.VMEM((tail,), jnp.int32),
                    pltpu.VMEM((tail, DIM), F32)]

    @functools.partial(
        pl.kernel, mesh=mesh,
        out_type=jax.ShapeDtypeStruct((N_EDGES, DIM), F32),
        scratch_types=scratch,
    )
    def k(v_hbm, dst_hbm, z_hbm, out_hbm, idx_v, loc_v, val_v, sem_i, sem_v,
          acc, *tail_bufs):
        cid = lax.axis_index("c")
        sid = lax.axis_index("s")
        tbase = sid * per_tile

        def start(off, s):
            pltpu.async_copy(dst_hbm.at[pl.ds(off, _WIN)], idx_v[s], sem_i[s])
            pltpu.async_copy(v_hbm.at[pl.ds(off, _WIN)], val_v[s], sem_v[s])

        def locs(nv, iv, lv, cbase):
            for kk in range(nv):
                dv = iv[pl.ds(kk * 16, 16)]
                loc = dv - cbase
                ok = (loc >= 0) & (loc < _EC_C)
                dummy = _EC_C + (dv & (_EC_DUMMY - 1))
                lv[pl.ds(kk * 16, 16)] = jnp.where(ok, loc, dummy)

        def finish(s, cbase):
            pltpu.make_async_copy(dst_hbm.at[pl.ds(0, _WIN)], idx_v[s],
                                  sem_i[s]).wait()
            pltpu.make_async_copy(v_hbm.at[pl.ds(0, _WIN)], val_v[s],
                                  sem_v[s]).wait()
            locs(_WIN // 16, idx_v[s], loc_v[s], cbase)
            pltpu.sync_copy(val_v[s], acc.at[loc_v[s]], add=True)

        def win_sync(off, w, nv, iv, lv, vv, cbase):
            pltpu.sync_copy(dst_hbm.at[pl.ds(off, w)], iv)
            pltpu.sync_copy(v_hbm.at[pl.ds(off, w)], vv)
            locs(nv, iv, lv, cbase)
            pltpu.sync_copy(vv, acc.at[lv], add=True)

        def writeout(cbase, csize):
            orows_tile = csize // NS
            pltpu.sync_copy(
                acc.at[pl.ds(sid * orows_tile, orows_tile)],
                out_hbm.at[pl.ds(cbase + sid * orows_tile, orows_tile)])

        for p in range(per_sc):
            chunk = cid * per_sc + p
            cbase = chunk * _EC_C
            pltpu.sync_copy(
                z_hbm.at[pl.ds(sid * zrows_tile, zrows_tile)],
                acc.at[pl.ds(sid * zrows_tile, zrows_tile)])
            plsc.subcore_barrier()

            start(tbase, 0)

            def body(kh, carry):
                w0 = tbase + 2 * kh * _WIN
                start(w0 + _WIN, 1)
                finish(0, cbase)

                @pl.when(kh < n_half - 1)
                def _pf():
                    start(w0 + 2 * _WIN, 0)

                finish(1, cbase)
                return carry

            lax.fori_loop(0, n_half, body, 0)
            if tail:
                win_sync(tbase + n_full * _WIN, tail, tail // 16,
                         tail_bufs[0], tail_bufs[1], tail_bufs[2], cbase)
            plsc.subcore_barrier()
            if p < per_sc - 1:
                writeout(cbase, _EC_C)
            else:
                last0 = _EC_CHUNKS[per_sc - 1]
                last1 = _EC_CHUNKS[NC * per_sc - 1]

                @pl.when(cid == 0)
                def _w0():
                    writeout(last0[0], last0[1])

                @pl.when(cid == 1)
                def _w1():
                    writeout(last1[0], last1[1])
            plsc.subcore_barrier()

    return k(v, dst, zeros)


def _sc_segsum_nodes(v, dst, zeros):
    e = v.shape[0]
    per_sc = e // NC
    per_tile = per_sc // NS
    n_full, tail = divmod(per_tile, _WIN)
    nrows = 10240
    zrows_tile = nrows // NS
    mesh = plsc.VectorSubcoreMesh(**_SC_MESH)

    scratch = [
        pltpu.VMEM((_WIN,), jnp.int32),
        pltpu.VMEM((_WIN, DIM), F32),
        pltpu.VMEM_SHARED((nrows, DIM), F32),
    ]
    if tail:
        scratch += [pltpu.VMEM((tail,), jnp.int32), pltpu.VMEM((tail, DIM), F32)]

    @functools.partial(
        pl.kernel, mesh=mesh,
        out_type=jax.ShapeDtypeStruct((NC * nrows, DIM), F32),
        scratch_types=scratch,
    )
    def k(v_hbm, dst_hbm, z_hbm, out_hbm, idx_v, val_v, acc, *tail_bufs):
        cid = lax.axis_index("c")
        sid = lax.axis_index("s")
        tbase = cid * per_sc + sid * per_tile

        pltpu.sync_copy(z_hbm.at[pl.ds(sid * zrows_tile, zrows_tile)],
                        acc.at[pl.ds(sid * zrows_tile, zrows_tile)])
        plsc.subcore_barrier()

        def win(off, w, iv, vv):
            pltpu.sync_copy(dst_hbm.at[pl.ds(off, w)], iv)
            pltpu.sync_copy(v_hbm.at[pl.ds(off, w)], vv)
            pltpu.sync_copy(vv, acc.at[iv], add=True)

        def body(iw, carry):
            win(tbase + iw * _WIN, _WIN, idx_v, val_v)
            return carry

        lax.fori_loop(0, n_full, body, 0)
        if tail:
            win(tbase + n_full * _WIN, tail, tail_bufs[0], tail_bufs[1])
        plsc.subcore_barrier()
        pltpu.sync_copy(
            acc.at[pl.ds(sid * zrows_tile, zrows_tile)],
            out_hbm.at[pl.ds(cid * nrows + sid * zrows_tile, zrows_tile)])

    return k(v, dst, zeros)


def kernel(h, rbf, sbf1, sbf2, idx_kj, idx_ji_1, idx_jj, idx_ji_2,
           edge_index, params):
    p = params
    i32 = jnp.int32
    j = edge_index[0].astype(i32)
    i = edge_index[1].astype(i32)
    idx_kj = idx_kj.astype(i32)
    idx_ji_1 = idx_ji_1.astype(i32)
    idx_jj = idx_jj.astype(i32)
    idx_ji_2 = idx_ji_2.astype(i32)

    def wb(layer):
        w, b = layer
        return w, b.reshape(1, DIM)

    wh, bh = wb(p['h_mlp'][0])
    wkj, bkj = wb(p['mlp_kj'][0])
    wj1, bj1 = wb(p['mlp_ji_1'][0])
    wjj, bjj = wb(p['mlp_jj'][0])
    wj2, bj2 = wb(p['mlp_ji_2'][0])
    s1w1, s1b1 = wb(p['mlp_sbf1'][0])
    s1w2, s1b2 = wb(p['mlp_sbf1'][1])
    s2w1, s2b1 = wb(p['mlp_sbf2'][0])
    s2w2, s2b2 = wb(p['mlp_sbf2'][1])

    zeros = jnp.zeros((_EC_ROWS, DIM), F32)

    hh = _tc_node1(h, wh, bh)
    hh_i = _sc_gather(hh, i)
    hh_j = _sc_gather(hh, j)

    t1, mji1 = _tc_edge1(
        hh_i, hh_j, rbf,
        wkj[:DIM], wkj[DIM:2 * DIM], wkj[2 * DIM:], bkj,
        wj1[:DIM], wj1[DIM:2 * DIM], wj1[2 * DIM:], bj1,
        p['lin_rbf1'])

    g1 = _sc_gather(t1, idx_kj)
    v1 = _tc_trip(sbf1, g1, s1w1, s1b1, s1w2, s1b2)
    agg1 = _sc_segsum_edges(v1, idx_ji_1, zeros)

    t2, mji2, r3 = _tc_edge2(mji1, agg1, rbf, wjj, bjj, p['lin_rbf2'],
                             wj2, bj2, p['lin_rbf_out'])

    g2 = _sc_gather(t2, idx_jj)
    v2 = _tc_trip(sbf2, g2, s2w1, s2b1, s2w2, s2b2)
    agg2 = _sc_segsum_edges(v2, idx_ji_2, zeros)

    m3 = _tc_edge3(mji2, agg2, r3)
    hparts = _sc_segsum_nodes(m3, i, zeros)
    pa = hparts[:N_NODES]
    pb = hparts[10240:10240 + N_NODES]

    weights = []
    for (w1, b1), (w2, b2) in [(p['res1'][0], p['res1'][1])]:
        weights += [w1, b1.reshape(1, DIM), w2, b2.reshape(1, DIM)]
    weights += [wh, bh]
    for key in ('res2', 'res3'):
        (w1, b1), (w2, b2) = p[key]
        weights += [w1, b1.reshape(1, DIM), w2, b2.reshape(1, DIM)]
    for w, b in p['y_mlp']:
        weights += [w, b.reshape(1, DIM)]
    wy, by = p['y_W']
    weights += [wy, by.reshape(1, 1)]

    h_out, y = _tc_node2(pa, pb, h, weights)
    return (h_out, y)


# speedup vs baseline: 1.4086x; 1.0576x over previous
import functools

import jax
import jax.numpy as jnp
from jax import lax
from jax.experimental import pallas as pl
from jax.experimental.pallas import tpu as pltpu
from jax.experimental.pallas import tpu_sc as plsc

DIM = 128
N_NODES = 10000
N_EDGES = 160000
N_TRIPLETS = 320000

NC = 2
NS = 16
NW = NC * NS

F32 = jnp.float32


def _silu(x):
    return x * jax.nn.sigmoid(x)


def _dot(a, b):
    return jnp.dot(a, b, preferred_element_type=F32)


def _row_spec(blk):
    return pl.BlockSpec((blk, DIM), lambda b: (b, 0))


def _w_spec(shape):
    return pl.BlockSpec(shape, lambda b: tuple(0 for _ in shape))


def _node1_body(h_ref, w_ref, b_ref, o_ref):
    o_ref[:] = _silu(_dot(h_ref[:], w_ref[:]) + b_ref[:])


def _tc_node1(h, w, b):
    n, blk = h.shape[0], 2000
    return pl.pallas_call(
        _node1_body,
        grid=(n // blk,),
        in_specs=[_row_spec(blk), _w_spec((DIM, DIM)), _w_spec((1, DIM))],
        out_specs=_row_spec(blk),
        out_shape=jax.ShapeDtypeStruct((n, DIM), F32),
    )(h, w, b)


def _edge1_body(hi, hj, rbf, wa1, wb1, wc1, b1, wa2, wb2, wc2, b2, l1,
                t1_o, mji1_o):
    x = rbf[:]
    mkj = _silu(_dot(hi[:], wa1[:]) + _dot(hj[:], wb1[:]) + _dot(x, wc1[:])
                + b1[:])
    t1_o[:] = mkj * _dot(x, l1[:])
    mji1_o[:] = _silu(_dot(hi[:], wa2[:]) + _dot(hj[:], wb2[:])
                      + _dot(x, wc2[:]) + b2[:])


def _tc_edge1(hi, hj, rbf, wa1, wb1, wc1, b1, wa2, wb2, wc2, b2, l1):
    n, blk = rbf.shape[0], 640
    ws = [_w_spec((DIM, DIM))] * 3 + [_w_spec((1, DIM))]
    return pl.pallas_call(
        _edge1_body,
        grid=(n // blk,),
        in_specs=[_row_spec(blk)] * 3 + ws + ws + [_w_spec((DIM, DIM))],
        out_specs=[_row_spec(blk)] * 2,
        out_shape=[jax.ShapeDtypeStruct((n, DIM), F32)] * 2,
    )(hi, hj, rbf, wa1, wb1, wc1, b1, wa2, wb2, wc2, b2, l1)


def _trip_body(sbf, g, w1, b1, w2, b2, v_o):
    x = _silu(_dot(sbf[:], w1[:]) + b1[:])
    x = _silu(_dot(x, w2[:]) + b2[:])
    v_o[:] = x * g[:]


def _tc_trip(sbf, g, w1, b1, w2, b2):
    n, blk = sbf.shape[0], 640
    return pl.pallas_call(
        _trip_body,
        grid=(n // blk,),
        in_specs=[_row_spec(blk)] * 2 + [_w_spec((DIM, DIM)), _w_spec((1, DIM)),
                                         _w_spec((DIM, DIM)), _w_spec((1, DIM))],
        out_specs=_row_spec(blk),
        out_shape=jax.ShapeDtypeStruct((n, DIM), F32),
    )(sbf, g, w1, b1, w2, b2)


def _edge2_body(mji1, agg1, rbf, wjj, bjj, l2, wji2, bji2, l3,
                t2_o, mji2_o, r3_o):
    m2 = mji1[:] + agg1[:]
    x = rbf[:]
    t2_o[:] = _silu(_dot(m2, wjj[:]) + bjj[:]) * _dot(x, l2[:])
    mji2_o[:] = _silu(_dot(m2, wji2[:]) + bji2[:])
    r3_o[:] = _dot(x, l3[:])


def _tc_edge2(mji1, agg1, rbf, wjj, bjj, l2, wji2, bji2, l3):
    n, blk = rbf.shape[0], 640
    return pl.pallas_call(
        _edge2_body,
        grid=(n // blk,),
        in_specs=[_row_spec(blk)] * 3
        + [_w_spec((DIM, DIM)), _w_spec((1, DIM)), _w_spec((DIM, DIM)),
           _w_spec((DIM, DIM)), _w_spec((1, DIM)), _w_spec((DIM, DIM))],
        out_specs=[_row_spec(blk)] * 3,
        out_shape=[jax.ShapeDtypeStruct((n, DIM), F32)] * 3,
    )(mji1, agg1, rbf, wjj, bjj, l2, wji2, bji2, l3)


def _edge3_body(mji2, agg2, r3, m3_o):
    m3_o[:] = r3[:] * (mji2[:] + agg2[:])


def _tc_edge3(mji2, agg2, r3):
    n, blk = mji2.shape[0], 2000
    return pl.pallas_call(
        _edge3_body,
        grid=(n // blk,),
        in_specs=[_row_spec(blk)] * 3,
        out_specs=_row_spec(blk),
        out_shape=jax.ShapeDtypeStruct((n, DIM), F32),
    )(mji2, agg2, r3)


def _node2_body(pa, pb, h,
                r1w1, r1b1, r1w2, r1b2,
                hw, hb,
                r2w1, r2b1, r2w2, r2b2,
                r3w1, r3b1, r3w2, r3b2,
                yw1, yb1, yw2, yb2, yw3, yb3,
                wy, by,
                h_o, y_o):
    def res(x, w1, b1, w2, b2):
        z = _silu(_dot(x, w1[:]) + b1[:])
        z = _silu(_dot(z, w2[:]) + b2[:])
        return z + x

    t = pa[:] + pb[:]
    t = res(t, r1w1, r1b1, r1w2, r1b2)
    t = _silu(_dot(t, hw[:]) + hb[:]) + h[:]
    t = res(t, r2w1, r2b1, r2w2, r2b2)
    t = res(t, r3w1, r3b1, r3w2, r3b2)
    h_o[:] = t
    z = _silu(_dot(t, yw1[:]) + yb1[:])
    z = _silu(_dot(z, yw2[:]) + yb2[:])
    z = _silu(_dot(z, yw3[:]) + yb3[:])
    y_o[:] = _dot(z, wy[:]) + by[:]


def _tc_node2(pa, pb, h, weights):
    n, blk = h.shape[0], 2000
    wspecs = []
    for w in weights:
        wspecs.append(_w_spec(w.shape))
    return pl.pallas_call(
        _node2_body,
        grid=(n // blk,),
        in_specs=[_row_spec(blk)] * 3 + wspecs,
        out_specs=[_row_spec(blk), pl.BlockSpec((blk, 1), lambda b: (b, 0))],
        out_shape=[jax.ShapeDtypeStruct((n, DIM), F32),
                   jax.ShapeDtypeStruct((n, 1), F32)],
    )(pa, pb, h, *weights)


_SC_MESH = dict(core_axis_name="c", subcore_axis_name="s",
                num_cores=NC, num_subcores=NS)
_WIN = 128


def _sc_gather(table, idx):
    b = idx.shape[0]
    per_w = b // NW
    n_full, tail = divmod(per_w, _WIN)
    mesh = plsc.VectorSubcoreMesh(**_SC_MESH)

    n_pairs = n_full // 2
    odd = n_full - 2 * n_pairs

    scratch = [
        [pltpu.VMEM((_WIN,), jnp.int32)] * 2,
        [pltpu.VMEM((_WIN, DIM), F32)] * 2,
        [pltpu.SemaphoreType.DMA] * 2,
    ]
    if tail:
        scratch += [pltpu.VMEM((tail,), jnp.int32), pltpu.VMEM((tail, DIM), F32)]

    @functools.partial(
        pl.kernel, mesh=mesh,
        out_type=jax.ShapeDtypeStruct((b, DIM), F32),
        scratch_types=scratch,
    )
    def k(table_hbm, idx_hbm, out_hbm, idx_v, rows_v, sem, *tail_bufs):
        wid = lax.axis_index("s") * NC + lax.axis_index("c")
        base = wid * per_w

        def gstart(off, s):
            pltpu.sync_copy(idx_hbm.at[pl.ds(off, _WIN)], idx_v[s])
            pltpu.async_copy(table_hbm.at[idx_v[s]], rows_v[s], sem[s])

        def gfinish(off, s):
            pltpu.make_async_copy(table_hbm.at[idx_v[s]], rows_v[s],
                                  sem[s]).wait()
            pltpu.sync_copy(rows_v[s], out_hbm.at[pl.ds(off, _WIN)])

        def win_sync(off, w, iv, rv):
            pltpu.sync_copy(idx_hbm.at[pl.ds(off, w)], iv)
            pltpu.async_copy(table_hbm.at[iv], rv, sem[0]).wait()
            pltpu.sync_copy(rv, out_hbm.at[pl.ds(off, w)])

        if n_pairs:
            gstart(base, 0)

            def body(kh, carry):
                w0 = base + 2 * kh * _WIN
                gstart(w0 + _WIN, 1)
                gfinish(w0, 0)

                @pl.when(kh < n_pairs - 1)
                def _pf():
                    gstart(w0 + 2 * _WIN, 0)

                gfinish(w0 + _WIN, 1)
                return carry

            lax.fori_loop(0, n_pairs, body, 0)
        if odd:
            win_sync(base + 2 * n_pairs * _WIN, _WIN, idx_v[0], rows_v[0])
        if tail:
            win_sync(base + n_full * _WIN, tail, tail_bufs[0], tail_bufs[1])

    return k(table, idx)


_EC_C = 10240
_EC_DUMMY = 256
_EC_ROWS = _EC_C + _EC_DUMMY
_EC_CHUNKS = [(b, min(_EC_C, N_EDGES - b)) for b in range(0, N_EDGES, _EC_C)]


def _sc_segsum_edges(v, dst, zeros):
    t = v.shape[0]
    per_sc = len(_EC_CHUNKS) // NC
    per_tile = t // NS
    n_full, tail = divmod(per_tile, _WIN)
    zrows_tile = _EC_ROWS // NS
    mesh = plsc.VectorSubcoreMesh(**_SC_MESH)

    n_half = n_full // 2
    assert n_full == 2 * n_half, "window count must be even for 2-deep ring"

    scratch = [
        [pltpu.VMEM((_WIN,), jnp.int32)] * 2,
        [pltpu.VMEM((_WIN,), jnp.int32)] * 2,
        [pltpu.VMEM((_WIN, DIM), F32)] * 2,
        [pltpu.SemaphoreType.DMA] * 2,
        [pltpu.SemaphoreType.DMA] * 2,
        pltpu.VMEM_SHARED((_EC_ROWS, DIM), F32),
    ]
    if tail:
        scratch += [pltpu.VMEM((tail,), jnp.int32),
                    pltpu.VMEM((tail,), jnp.int32),
                    pltpu.VMEM((tail, DIM), F32)]

    @functools.partial(
        pl.kernel, mesh=mesh,
        out_type=jax.ShapeDtypeStruct((N_EDGES, DIM), F32),
        scratch_types=scratch,
    )
    def k(v_hbm, dst_hbm, z_hbm, out_hbm, idx_v, loc_v, val_v, sem_i, sem_v,
          acc, *tail_bufs):
        cid = lax.axis_index("c")
        sid = lax.axis_index("s")
        tbase = sid * per_tile

        def start(off, s):
            pltpu.async_copy(dst_hbm.at[pl.ds(off, _WIN)], idx_v[s], sem_i[s])
            pltpu.async_copy(v_hbm.at[pl.ds(off, _WIN)], val_v[s], sem_v[s])

        def locs(nv, iv, lv, cbase):
            for kk in range(nv):
                dv = iv[pl.ds(kk * 16, 16)]
                loc = dv - cbase
                ok = (loc >= 0) & (loc < _EC_C)
                dummy = _EC_C + (dv & (_EC_DUMMY - 1))
                lv[pl.ds(kk * 16, 16)] = jnp.where(ok, loc, dummy)

        def finish(s, cbase):
            pltpu.make_async_copy(dst_hbm.at[pl.ds(0, _WIN)], idx_v[s],
                                  sem_i[s]).wait()
            pltpu.make_async_copy(v_hbm.at[pl.ds(0, _WIN)], val_v[s],
                                  sem_v[s]).wait()
            locs(_WIN // 16, idx_v[s], loc_v[s], cbase)
            pltpu.sync_copy(val_v[s], acc.at[loc_v[s]], add=True)

        def win_sync(off, w, nv, iv, lv, vv, cbase):
            pltpu.sync_copy(dst_hbm.at[pl.ds(off, w)], iv)
            pltpu.sync_copy(v_hbm.at[pl.ds(off, w)], vv)
            locs(nv, iv, lv, cbase)
            pltpu.sync_copy(vv, acc.at[lv], add=True)

        def writeout(cbase, csize):
            orows_tile = csize // NS
            pltpu.sync_copy(
                acc.at[pl.ds(sid * orows_tile, orows_tile)],
                out_hbm.at[pl.ds(cbase + sid * orows_tile, orows_tile)])

        for p in range(per_sc):
            chunk = cid * per_sc + p
            cbase = chunk * _EC_C
            pltpu.sync_copy(
                z_hbm.at[pl.ds(sid * zrows_tile, zrows_tile)],
                acc.at[pl.ds(sid * zrows_tile, zrows_tile)])
            plsc.subcore_barrier()

            start(tbase, 0)

            def body(kh, carry):
                w0 = tbase + 2 * kh * _WIN
                start(w0 + _WIN, 1)
                finish(0, cbase)

                @pl.when(kh < n_half - 1)
                def _pf():
                    start(w0 + 2 * _WIN, 0)

                finish(1, cbase)
                return carry

            lax.fori_loop(0, n_half, body, 0)
            if tail:
                win_sync(tbase + n_full * _WIN, tail, tail // 16,
                         tail_bufs[0], tail_bufs[1], tail_bufs[2], cbase)
            plsc.subcore_barrier()
            if p < per_sc - 1:
                writeout(cbase, _EC_C)
            else:
                last0 = _EC_CHUNKS[per_sc - 1]
                last1 = _EC_CHUNKS[NC * per_sc - 1]

                @pl.when(cid == 0)
                def _w0():
                    writeout(last0[0], last0[1])

                @pl.when(cid == 1)
                def _w1():
                    writeout(last1[0], last1[1])
            plsc.subcore_barrier()

    return k(v, dst, zeros)


def _sc_segsum_nodes(v, dst, zeros):
    e = v.shape[0]
    per_sc = e // NC
    per_tile = per_sc // NS
    n_full, tail = divmod(per_tile, _WIN)
    nrows = 10240
    zrows_tile = nrows // NS
    mesh = plsc.VectorSubcoreMesh(**_SC_MESH)

    n_pairs = n_full // 2
    odd = n_full - 2 * n_pairs

    scratch = [
        [pltpu.VMEM((_WIN,), jnp.int32)] * 2,
        [pltpu.VMEM((_WIN, DIM), F32)] * 2,
        [pltpu.SemaphoreType.DMA] * 2,
        [pltpu.SemaphoreType.DMA] * 2,
        pltpu.VMEM_SHARED((nrows, DIM), F32),
    ]
    if tail:
        scratch += [pltpu.VMEM((tail,), jnp.int32), pltpu.VMEM((tail, DIM), F32)]

    @functools.partial(
        pl.kernel, mesh=mesh,
        out_type=jax.ShapeDtypeStruct((NC * nrows, DIM), F32),
        scratch_types=scratch,
    )
    def k(v_hbm, dst_hbm, z_hbm, out_hbm, idx_v, val_v, sem_i, sem_v, acc,
          *tail_bufs):
        cid = lax.axis_index("c")
        sid = lax.axis_index("s")
        tbase = cid * per_sc + sid * per_tile

        pltpu.sync_copy(z_hbm.at[pl.ds(sid * zrows_tile, zrows_tile)],
                        acc.at[pl.ds(sid * zrows_tile, zrows_tile)])
        plsc.subcore_barrier()

        def start(off, s):
            pltpu.async_copy(dst_hbm.at[pl.ds(off, _WIN)], idx_v[s], sem_i[s])
            pltpu.async_copy(v_hbm.at[pl.ds(off, _WIN)], val_v[s], sem_v[s])

        def finish(s):
            pltpu.make_async_copy(dst_hbm.at[pl.ds(0, _WIN)], idx_v[s],
                                  sem_i[s]).wait()
            pltpu.make_async_copy(v_hbm.at[pl.ds(0, _WIN)], val_v[s],
                                  sem_v[s]).wait()
            pltpu.sync_copy(val_v[s], acc.at[idx_v[s]], add=True)

        def win_sync(off, w, iv, vv):
            pltpu.sync_copy(dst_hbm.at[pl.ds(off, w)], iv)
            pltpu.sync_copy(v_hbm.at[pl.ds(off, w)], vv)
            pltpu.sync_copy(vv, acc.at[iv], add=True)

        if n_pairs:
            start(tbase, 0)

            def body(kh, carry):
                w0 = tbase + 2 * kh * _WIN
                start(w0 + _WIN, 1)
                finish(0)

                @pl.when(kh < n_pairs - 1)
                def _pf():
                    start(w0 + 2 * _WIN, 0)

                finish(1)
                return carry

            lax.fori_loop(0, n_pairs, body, 0)
        if odd:
            win_sync(tbase + 2 * n_pairs * _WIN, _WIN, idx_v[0], val_v[0])
        if tail:
            win_sync(tbase + n_full * _WIN, tail, tail_bufs[0], tail_bufs[1])
        plsc.subcore_barrier()
        pltpu.sync_copy(
            acc.at[pl.ds(sid * zrows_tile, zrows_tile)],
            out_hbm.at[pl.ds(cid * nrows + sid * zrows_tile, zrows_tile)])

    return k(v, dst, zeros)


def kernel(h, rbf, sbf1, sbf2, idx_kj, idx_ji_1, idx_jj, idx_ji_2,
           edge_index, params):
    p = params
    i32 = jnp.int32
    j = edge_index[0].astype(i32)
    i = edge_index[1].astype(i32)
    idx_kj = idx_kj.astype(i32)
    idx_ji_1 = idx_ji_1.astype(i32)
    idx_jj = idx_jj.astype(i32)
    idx_ji_2 = idx_ji_2.astype(i32)

    def wb(layer):
        w, b = layer
        return w, b.reshape(1, DIM)

    wh, bh = wb(p['h_mlp'][0])
    wkj, bkj = wb(p['mlp_kj'][0])
    wj1, bj1 = wb(p['mlp_ji_1'][0])
    wjj, bjj = wb(p['mlp_jj'][0])
    wj2, bj2 = wb(p['mlp_ji_2'][0])
    s1w1, s1b1 = wb(p['mlp_sbf1'][0])
    s1w2, s1b2 = wb(p['mlp_sbf1'][1])
    s2w1, s2b1 = wb(p['mlp_sbf2'][0])
    s2w2, s2b2 = wb(p['mlp_sbf2'][1])

    zeros = jnp.zeros((_EC_ROWS, DIM), F32)

    hh = _tc_node1(h, wh, bh)
    hh_i = _sc_gather(hh, i)
    hh_j = _sc_gather(hh, j)

    t1, mji1 = _tc_edge1(
        hh_i, hh_j, rbf,
        wkj[:DIM], wkj[DIM:2 * DIM], wkj[2 * DIM:], bkj,
        wj1[:DIM], wj1[DIM:2 * DIM], wj1[2 * DIM:], bj1,
        p['lin_rbf1'])

    g1 = _sc_gather(t1, idx_kj)
    v1 = _tc_trip(sbf1, g1, s1w1, s1b1, s1w2, s1b2)
    agg1 = _sc_segsum_edges(v1, idx_ji_1, zeros)

    t2, mji2, r3 = _tc_edge2(mji1, agg1, rbf, wjj, bjj, p['lin_rbf2'],
                             wj2, bj2, p['lin_rbf_out'])

    g2 = _sc_gather(t2, idx_jj)
    v2 = _tc_trip(sbf2, g2, s2w1, s2b1, s2w2, s2b2)
    agg2 = _sc_segsum_edges(v2, idx_ji_2, zeros)

    m3 = _tc_edge3(mji2, agg2, r3)
    hparts = _sc_segsum_nodes(m3, i, zeros)
    pa = hparts[:N_NODES]
    pb = hparts[10240:10240 + N_NODES]

    weights = []
    for (w1, b1), (w2, b2) in [(p['res1'][0], p['res1'][1])]:
        weights += [w1, b1.reshape(1, DIM), w2, b2.reshape(1, DIM)]
    weights += [wh, bh]
    for key in ('res2', 'res3'):
        (w1, b1), (w2, b2) = p[key]
        weights += [w1, b1.reshape(1, DIM), w2, b2.reshape(1, DIM)]
    for w, b in p['y_mlp']:
        weights += [w, b.reshape(1, DIM)]
    wy, by = p['y_W']
    weights += [wy, by.reshape(1, 1)]

    h_out, y = _tc_node2(pa, pb, h, weights)
    return (h_out, y)


# speedup vs baseline: 1.5417x; 1.0945x over previous
import functools

import jax
import jax.numpy as jnp
from jax import lax
from jax.experimental import pallas as pl
from jax.experimental.pallas import tpu as pltpu
from jax.experimental.pallas import tpu_sc as plsc

DIM = 128
N_NODES = 10000
N_EDGES = 160000
N_TRIPLETS = 320000

NC = 2
NS = 16
NW = NC * NS

F32 = jnp.float32


def _silu(x):
    return x * jax.nn.sigmoid(x)


def _dot(a, b):
    return jnp.dot(a, b, preferred_element_type=F32)


def _row_spec(blk):
    return pl.BlockSpec((blk, DIM), lambda b: (b, 0))


def _w_spec(shape):
    return pl.BlockSpec(shape, lambda b: tuple(0 for _ in shape))


def _node1_body(h_ref, w_ref, b_ref, o_ref):
    o_ref[:] = _silu(_dot(h_ref[:], w_ref[:]) + b_ref[:])


def _tc_node1(h, w, b):
    n, blk = h.shape[0], 2000
    return pl.pallas_call(
        _node1_body,
        grid=(n // blk,),
        in_specs=[_row_spec(blk), _w_spec((DIM, DIM)), _w_spec((1, DIM))],
        out_specs=_row_spec(blk),
        out_shape=jax.ShapeDtypeStruct((n, DIM), F32),
    )(h, w, b)


def _edge1_body(hi, hj, rbf, wa1, wb1, wc1, b1, wa2, wb2, wc2, b2, l1,
                t1_o, mji1_o):
    x = rbf[:]
    mkj = _silu(_dot(hi[:], wa1[:]) + _dot(hj[:], wb1[:]) + _dot(x, wc1[:])
                + b1[:])
    t1_o[:] = mkj * _dot(x, l1[:])
    mji1_o[:] = _silu(_dot(hi[:], wa2[:]) + _dot(hj[:], wb2[:])
                      + _dot(x, wc2[:]) + b2[:])


def _tc_edge1(hi, hj, rbf, wa1, wb1, wc1, b1, wa2, wb2, wc2, b2, l1):
    n, blk = rbf.shape[0], 1280
    ws = [_w_spec((DIM, DIM))] * 3 + [_w_spec((1, DIM))]
    return pl.pallas_call(
        _edge1_body,
        grid=(n // blk,),
        in_specs=[_row_spec(blk)] * 3 + ws + ws + [_w_spec((DIM, DIM))],
        out_specs=[_row_spec(blk)] * 2,
        out_shape=[jax.ShapeDtypeStruct((n, DIM), F32)] * 2,
    )(hi, hj, rbf, wa1, wb1, wc1, b1, wa2, wb2, wc2, b2, l1)


def _trip_body(sbf, g, w1, b1, w2, b2, v_o):
    x = _silu(_dot(sbf[:], w1[:]) + b1[:])
    x = _silu(_dot(x, w2[:]) + b2[:])
    v_o[:] = x * g[:]


def _tc_trip(sbf, g, w1, b1, w2, b2):
    n, blk = sbf.shape[0], 1280
    return pl.pallas_call(
        _trip_body,
        grid=(n // blk,),
        in_specs=[_row_spec(blk)] * 2 + [_w_spec((DIM, DIM)), _w_spec((1, DIM)),
                                         _w_spec((DIM, DIM)), _w_spec((1, DIM))],
        out_specs=_row_spec(blk),
        out_shape=jax.ShapeDtypeStruct((n, DIM), F32),
    )(sbf, g, w1, b1, w2, b2)


def _edge2_body(mji1, agg1, rbf, wjj, bjj, l2, wji2, bji2, l3,
                t2_o, mji2_o, r3_o):
    m2 = mji1[:] + agg1[:]
    x = rbf[:]
    t2_o[:] = _silu(_dot(m2, wjj[:]) + bjj[:]) * _dot(x, l2[:])
    mji2_o[:] = _silu(_dot(m2, wji2[:]) + bji2[:])
    r3_o[:] = _dot(x, l3[:])


def _tc_edge2(mji1, agg1, rbf, wjj, bjj, l2, wji2, bji2, l3):
    n, blk = rbf.shape[0], 1280
    return pl.pallas_call(
        _edge2_body,
        grid=(n // blk,),
        in_specs=[_row_spec(blk)] * 3
        + [_w_spec((DIM, DIM)), _w_spec((1, DIM)), _w_spec((DIM, DIM)),
           _w_spec((DIM, DIM)), _w_spec((1, DIM)), _w_spec((DIM, DIM))],
        out_specs=[_row_spec(blk)] * 3,
        out_shape=[jax.ShapeDtypeStruct((n, DIM), F32)] * 3,
    )(mji1, agg1, rbf, wjj, bjj, l2, wji2, bji2, l3)


def _edge3_body(mji2, agg2, r3, m3_o):
    m3_o[:] = r3[:] * (mji2[:] + agg2[:])


def _tc_edge3(mji2, agg2, r3):
    n, blk = mji2.shape[0], 2000
    return pl.pallas_call(
        _edge3_body,
        grid=(n // blk,),
        in_specs=[_row_spec(blk)] * 3,
        out_specs=_row_spec(blk),
        out_shape=jax.ShapeDtypeStruct((n, DIM), F32),
    )(mji2, agg2, r3)


def _node2_body(pa, pb, h,
                r1w1, r1b1, r1w2, r1b2,
                hw, hb,
                r2w1, r2b1, r2w2, r2b2,
                r3w1, r3b1, r3w2, r3b2,
                yw1, yb1, yw2, yb2, yw3, yb3,
                wy, by,
                h_o, y_o):
    def res(x, w1, b1, w2, b2):
        z = _silu(_dot(x, w1[:]) + b1[:])
        z = _silu(_dot(z, w2[:]) + b2[:])
        return z + x

    t = pa[:] + pb[:]
    t = res(t, r1w1, r1b1, r1w2, r1b2)
    t = _silu(_dot(t, hw[:]) + hb[:]) + h[:]
    t = res(t, r2w1, r2b1, r2w2, r2b2)
    t = res(t, r3w1, r3b1, r3w2, r3b2)
    h_o[:] = t
    z = _silu(_dot(t, yw1[:]) + yb1[:])
    z = _silu(_dot(z, yw2[:]) + yb2[:])
    z = _silu(_dot(z, yw3[:]) + yb3[:])
    y_o[:] = _dot(z, wy[:]) + by[:]


def _tc_node2(pa, pb, h, weights):
    n, blk = h.shape[0], 2000
    wspecs = []
    for w in weights:
        wspecs.append(_w_spec(w.shape))
    return pl.pallas_call(
        _node2_body,
        grid=(n // blk,),
        in_specs=[_row_spec(blk)] * 3 + wspecs,
        out_specs=[_row_spec(blk), pl.BlockSpec((blk, 1), lambda b: (b, 0))],
        out_shape=[jax.ShapeDtypeStruct((n, DIM), F32),
                   jax.ShapeDtypeStruct((n, 1), F32)],
    )(pa, pb, h, *weights)


_SC_MESH = dict(core_axis_name="c", subcore_axis_name="s",
                num_cores=NC, num_subcores=NS)
_WIN = 128


def _sc_gather(table, idx):
    b = idx.shape[0]
    per_w = b // NW
    n_full, tail = divmod(per_w, _WIN)
    mesh = plsc.VectorSubcoreMesh(**_SC_MESH)

    n_pairs = n_full // 2
    odd = n_full - 2 * n_pairs

    scratch = [
        [pltpu.VMEM((_WIN,), jnp.int32)] * 2,
        [pltpu.VMEM((_WIN, DIM), F32)] * 2,
        [pltpu.SemaphoreType.DMA] * 2,
    ]
    if tail:
        scratch += [pltpu.VMEM((tail,), jnp.int32), pltpu.VMEM((tail, DIM), F32)]

    @functools.partial(
        pl.kernel, mesh=mesh,
        out_type=jax.ShapeDtypeStruct((b, DIM), F32),
        scratch_types=scratch,
    )
    def k(table_hbm, idx_hbm, out_hbm, idx_v, rows_v, sem, *tail_bufs):
        wid = lax.axis_index("s") * NC + lax.axis_index("c")
        base = wid * per_w

        def gstart(off, s):
            pltpu.sync_copy(idx_hbm.at[pl.ds(off, _WIN)], idx_v[s])
            pltpu.async_copy(table_hbm.at[idx_v[s]], rows_v[s], sem[s])

        def gfinish(off, s):
            pltpu.make_async_copy(table_hbm.at[idx_v[s]], rows_v[s],
                                  sem[s]).wait()
            pltpu.sync_copy(rows_v[s], out_hbm.at[pl.ds(off, _WIN)])

        def win_sync(off, w, iv, rv):
            pltpu.sync_copy(idx_hbm.at[pl.ds(off, w)], iv)
            pltpu.async_copy(table_hbm.at[iv], rv, sem[0]).wait()
            pltpu.sync_copy(rv, out_hbm.at[pl.ds(off, w)])

        if n_pairs:
            gstart(base, 0)

            def body(kh, carry):
                w0 = base + 2 * kh * _WIN
                gstart(w0 + _WIN, 1)
                gfinish(w0, 0)

                @pl.when(kh < n_pairs - 1)
                def _pf():
                    gstart(w0 + 2 * _WIN, 0)

                gfinish(w0 + _WIN, 1)
                return carry

            lax.fori_loop(0, n_pairs, body, 0)
        if odd:
            win_sync(base + 2 * n_pairs * _WIN, _WIN, idx_v[0], rows_v[0])
        if tail:
            win_sync(base + n_full * _WIN, tail, tail_bufs[0], tail_bufs[1])

    return k(table, idx)


_EC_C = 10240
_EC_DUMMY = 256
_EC_ROWS = _EC_C + _EC_DUMMY
_EC_CHUNKS = [(b, min(_EC_C, N_EDGES - b)) for b in range(0, N_EDGES, _EC_C)]


def _sc_segsum_edges(v, dst, zeros):
    t = v.shape[0]
    per_sc = len(_EC_CHUNKS) // NC
    per_tile = t // NS
    n_full, tail = divmod(per_tile, _WIN)
    zrows_tile = _EC_ROWS // NS
    mesh = plsc.VectorSubcoreMesh(**_SC_MESH)

    n_half = n_full // 2
    assert n_full == 2 * n_half, "window count must be even for 2-deep ring"

    scratch = [
        [pltpu.VMEM((_WIN,), jnp.int32)] * 2,
        [pltpu.VMEM((_WIN,), jnp.int32)] * 2,
        [pltpu.VMEM((_WIN, DIM), F32)] * 2,
        [pltpu.SemaphoreType.DMA] * 2,
        [pltpu.SemaphoreType.DMA] * 2,
        pltpu.VMEM_SHARED((_EC_ROWS, DIM), F32),
    ]
    if tail:
        scratch += [pltpu.VMEM((tail,), jnp.int32),
                    pltpu.VMEM((tail,), jnp.int32),
                    pltpu.VMEM((tail, DIM), F32)]

    @functools.partial(
        pl.kernel, mesh=mesh,
        out_type=jax.ShapeDtypeStruct((N_EDGES, DIM), F32),
        scratch_types=scratch,
    )
    def k(v_hbm, dst_hbm, z_hbm, out_hbm, idx_v, loc_v, val_v, sem_i, sem_v,
          acc, *tail_bufs):
        cid = lax.axis_index("c")
        sid = lax.axis_index("s")
        tbase = sid * per_tile

        def start(off, s):
            pltpu.async_copy(dst_hbm.at[pl.ds(off, _WIN)], idx_v[s], sem_i[s])
            pltpu.async_copy(v_hbm.at[pl.ds(off, _WIN)], val_v[s], sem_v[s])

        def locs(nv, iv, lv, cbase):
            for kk in range(nv):
                dv = iv[pl.ds(kk * 16, 16)]
                loc = dv - cbase
                ok = (loc >= 0) & (loc < _EC_C)
                dummy = _EC_C + (dv & (_EC_DUMMY - 1))
                lv[pl.ds(kk * 16, 16)] = jnp.where(ok, loc, dummy)

        def finish(s, cbase):
            pltpu.make_async_copy(dst_hbm.at[pl.ds(0, _WIN)], idx_v[s],
                                  sem_i[s]).wait()
            pltpu.make_async_copy(v_hbm.at[pl.ds(0, _WIN)], val_v[s],
                                  sem_v[s]).wait()
            locs(_WIN // 16, idx_v[s], loc_v[s], cbase)
            pltpu.sync_copy(val_v[s], acc.at[loc_v[s]], add=True)

        def win_sync(off, w, nv, iv, lv, vv, cbase):
            pltpu.sync_copy(dst_hbm.at[pl.ds(off, w)], iv)
            pltpu.sync_copy(v_hbm.at[pl.ds(off, w)], vv)
            locs(nv, iv, lv, cbase)
            pltpu.sync_copy(vv, acc.at[lv], add=True)

        def writeout(cbase, csize):
            orows_tile = csize // NS
            pltpu.sync_copy(
                acc.at[pl.ds(sid * orows_tile, orows_tile)],
                out_hbm.at[pl.ds(cbase + sid * orows_tile, orows_tile)])

        for p in range(per_sc):
            chunk = cid * per_sc + p
            cbase = chunk * _EC_C
            pltpu.sync_copy(
                z_hbm.at[pl.ds(sid * zrows_tile, zrows_tile)],
                acc.at[pl.ds(sid * zrows_tile, zrows_tile)])
            plsc.subcore_barrier()

            start(tbase, 0)

            def body(kh, carry):
                w0 = tbase + 2 * kh * _WIN
                start(w0 + _WIN, 1)
                finish(0, cbase)

                @pl.when(kh < n_half - 1)
                def _pf():
                    start(w0 + 2 * _WIN, 0)

                finish(1, cbase)
                return carry

            lax.fori_loop(0, n_half, body, 0)
            if tail:
                win_sync(tbase + n_full * _WIN, tail, tail // 16,
                         tail_bufs[0], tail_bufs[1], tail_bufs[2], cbase)
            plsc.subcore_barrier()
            if p < per_sc - 1:
                writeout(cbase, _EC_C)
            else:
                last0 = _EC_CHUNKS[per_sc - 1]
                last1 = _EC_CHUNKS[NC * per_sc - 1]

                @pl.when(cid == 0)
                def _w0():
                    writeout(last0[0], last0[1])

                @pl.when(cid == 1)
                def _w1():
                    writeout(last1[0], last1[1])
            plsc.subcore_barrier()

    return k(v, dst, zeros)


def _sc_segsum_nodes(v, dst, zeros):
    e = v.shape[0]
    per_sc = e // NC
    per_tile = per_sc // NS
    n_full, tail = divmod(per_tile, _WIN)
    nrows = 10240
    zrows_tile = nrows // NS
    mesh = plsc.VectorSubcoreMesh(**_SC_MESH)

    n_pairs = n_full // 2
    odd = n_full - 2 * n_pairs

    scratch = [
        [pltpu.VMEM((_WIN,), jnp.int32)] * 2,
        [pltpu.VMEM((_WIN, DIM), F32)] * 2,
        [pltpu.SemaphoreType.DMA] * 2,
        [pltpu.SemaphoreType.DMA] * 2,
        pltpu.VMEM_SHARED((nrows, DIM), F32),
    ]
    if tail:
        scratch += [pltpu.VMEM((tail,), jnp.int32), pltpu.VMEM((tail, DIM), F32)]

    @functools.partial(
        pl.kernel, mesh=mesh,
        out_type=jax.ShapeDtypeStruct((NC * nrows, DIM), F32),
        scratch_types=scratch,
    )
    def k(v_hbm, dst_hbm, z_hbm, out_hbm, idx_v, val_v, sem_i, sem_v, acc,
          *tail_bufs):
        cid = lax.axis_index("c")
        sid = lax.axis_index("s")
        tbase = cid * per_sc + sid * per_tile

        pltpu.sync_copy(z_hbm.at[pl.ds(sid * zrows_tile, zrows_tile)],
                        acc.at[pl.ds(sid * zrows_tile, zrows_tile)])
        plsc.subcore_barrier()

        def start(off, s):
            pltpu.async_copy(dst_hbm.at[pl.ds(off, _WIN)], idx_v[s], sem_i[s])
            pltpu.async_copy(v_hbm.at[pl.ds(off, _WIN)], val_v[s], sem_v[s])

        def finish(s):
            pltpu.make_async_copy(dst_hbm.at[pl.ds(0, _WIN)], idx_v[s],
                                  sem_i[s]).wait()
            pltpu.make_async_copy(v_hbm.at[pl.ds(0, _WIN)], val_v[s],
                                  sem_v[s]).wait()
            pltpu.sync_copy(val_v[s], acc.at[idx_v[s]], add=True)

        def win_sync(off, w, iv, vv):
            pltpu.sync_copy(dst_hbm.at[pl.ds(off, w)], iv)
            pltpu.sync_copy(v_hbm.at[pl.ds(off, w)], vv)
            pltpu.sync_copy(vv, acc.at[iv], add=True)

        if n_pairs:
            start(tbase, 0)

            def body(kh, carry):
                w0 = tbase + 2 * kh * _WIN
                start(w0 + _WIN, 1)
                finish(0)

                @pl.when(kh < n_pairs - 1)
                def _pf():
                    start(w0 + 2 * _WIN, 0)

                finish(1)
                return carry

            lax.fori_loop(0, n_pairs, body, 0)
        if odd:
            win_sync(tbase + 2 * n_pairs * _WIN, _WIN, idx_v[0], val_v[0])
        if tail:
            win_sync(tbase + n_full * _WIN, tail, tail_bufs[0], tail_bufs[1])
        plsc.subcore_barrier()
        pltpu.sync_copy(
            acc.at[pl.ds(sid * zrows_tile, zrows_tile)],
            out_hbm.at[pl.ds(cid * nrows + sid * zrows_tile, zrows_tile)])

    return k(v, dst, zeros)


def kernel(h, rbf, sbf1, sbf2, idx_kj, idx_ji_1, idx_jj, idx_ji_2,
           edge_index, params):
    p = params
    i32 = jnp.int32
    j = edge_index[0].astype(i32)
    i = edge_index[1].astype(i32)
    idx_kj = idx_kj.astype(i32)
    idx_ji_1 = idx_ji_1.astype(i32)
    idx_jj = idx_jj.astype(i32)
    idx_ji_2 = idx_ji_2.astype(i32)

    def wb(layer):
        w, b = layer
        return w, b.reshape(1, DIM)

    wh, bh = wb(p['h_mlp'][0])
    wkj, bkj = wb(p['mlp_kj'][0])
    wj1, bj1 = wb(p['mlp_ji_1'][0])
    wjj, bjj = wb(p['mlp_jj'][0])
    wj2, bj2 = wb(p['mlp_ji_2'][0])
    s1w1, s1b1 = wb(p['mlp_sbf1'][0])
    s1w2, s1b2 = wb(p['mlp_sbf1'][1])
    s2w1, s2b1 = wb(p['mlp_sbf2'][0])
    s2w2, s2b2 = wb(p['mlp_sbf2'][1])

    zeros = jnp.zeros((_EC_ROWS, DIM), F32)

    hh = _tc_node1(h, wh, bh)
    hh_i = _sc_gather(hh, i)
    hh_j = _sc_gather(hh, j)

    t1, mji1 = _tc_edge1(
        hh_i, hh_j, rbf,
        wkj[:DIM], wkj[DIM:2 * DIM], wkj[2 * DIM:], bkj,
        wj1[:DIM], wj1[DIM:2 * DIM], wj1[2 * DIM:], bj1,
        p['lin_rbf1'])

    g1 = _sc_gather(t1, idx_kj)
    v1 = _tc_trip(sbf1, g1, s1w1, s1b1, s1w2, s1b2)
    agg1 = _sc_segsum_edges(v1, idx_ji_1, zeros)

    t2, mji2, r3 = _tc_edge2(mji1, agg1, rbf, wjj, bjj, p['lin_rbf2'],
                             wj2, bj2, p['lin_rbf_out'])

    g2 = _sc_gather(t2, idx_jj)
    v2 = _tc_trip(sbf2, g2, s2w1, s2b1, s2w2, s2b2)
    agg2 = _sc_segsum_edges(v2, idx_ji_2, zeros)

    m3 = _tc_edge3(mji2, agg2, r3)
    hparts = _sc_segsum_nodes(m3, i, zeros)
    pa = hparts[:N_NODES]
    pb = hparts[10240:10240 + N_NODES]

    weights = []
    for (w1, b1), (w2, b2) in [(p['res1'][0], p['res1'][1])]:
        weights += [w1, b1.reshape(1, DIM), w2, b2.reshape(1, DIM)]
    weights += [wh, bh]
    for key in ('res2', 'res3'):
        (w1, b1), (w2, b2) = p[key]
        weights += [w1, b1.reshape(1, DIM), w2, b2.reshape(1, DIM)]
    for w, b in p['y_mlp']:
        weights += [w, b.reshape(1, DIM)]
    wy, by = p['y_W']
    weights += [wy, by.reshape(1, 1)]

    h_out, y = _tc_node2(pa, pb, h, weights)
    return (h_out, y)


# speedup vs baseline: 1.6458x; 1.0675x over previous
import functools

import jax
import jax.numpy as jnp
from jax import lax
from jax.experimental import pallas as pl
from jax.experimental.pallas import tpu as pltpu
from jax.experimental.pallas import tpu_sc as plsc

DIM = 128
N_NODES = 10000
N_EDGES = 160000
N_TRIPLETS = 320000

NC = 2
NS = 16
NW = NC * NS

F32 = jnp.float32


def _silu(x):
    return x * jax.nn.sigmoid(x)


def _dot(a, b):
    return jnp.dot(a, b, preferred_element_type=F32)


def _row_spec(blk):
    return pl.BlockSpec((blk, DIM), lambda b: (b, 0))


def _w_spec(shape):
    return pl.BlockSpec(shape, lambda b: tuple(0 for _ in shape))


def _node1_body(h_ref, w_ref, b_ref, o_ref):
    o_ref[:] = _silu(_dot(h_ref[:], w_ref[:]) + b_ref[:])


def _tc_node1(h, w, b):
    n, blk = h.shape[0], 2000
    return pl.pallas_call(
        _node1_body,
        grid=(n // blk,),
        in_specs=[_row_spec(blk), _w_spec((DIM, DIM)), _w_spec((1, DIM))],
        out_specs=_row_spec(blk),
        out_shape=jax.ShapeDtypeStruct((n, DIM), F32),
    )(h, w, b)


def _edge1_body(hi, hj, rbf, wa1, wb1, wc1, b1, wa2, wb2, wc2, b2, l1,
                t1_o, mji1_o):
    x = rbf[:]
    mkj = _silu(_dot(hi[:], wa1[:]) + _dot(hj[:], wb1[:]) + _dot(x, wc1[:])
                + b1[:])
    t1_o[:] = mkj * _dot(x, l1[:])
    mji1_o[:] = _silu(_dot(hi[:], wa2[:]) + _dot(hj[:], wb2[:])
                      + _dot(x, wc2[:]) + b2[:])


def _tc_edge1(hi, hj, rbf, wa1, wb1, wc1, b1, wa2, wb2, wc2, b2, l1):
    n, blk = rbf.shape[0], 3200
    ws = [_w_spec((DIM, DIM))] * 3 + [_w_spec((1, DIM))]
    return pl.pallas_call(
        _edge1_body,
        grid=(n // blk,),
        in_specs=[_row_spec(blk)] * 3 + ws + ws + [_w_spec((DIM, DIM))],
        out_specs=[_row_spec(blk)] * 2,
        out_shape=[jax.ShapeDtypeStruct((n, DIM), F32)] * 2,
    )(hi, hj, rbf, wa1, wb1, wc1, b1, wa2, wb2, wc2, b2, l1)


def _trip_body(sbf, g, w1, b1, w2, b2, v_o):
    x = _silu(_dot(sbf[:], w1[:]) + b1[:])
    x = _silu(_dot(x, w2[:]) + b2[:])
    v_o[:] = x * g[:]


def _tc_trip(sbf, g, w1, b1, w2, b2):
    n, blk = sbf.shape[0], 3200
    return pl.pallas_call(
        _trip_body,
        grid=(n // blk,),
        in_specs=[_row_spec(blk)] * 2 + [_w_spec((DIM, DIM)), _w_spec((1, DIM)),
                                         _w_spec((DIM, DIM)), _w_spec((1, DIM))],
        out_specs=_row_spec(blk),
        out_shape=jax.ShapeDtypeStruct((n, DIM), F32),
    )(sbf, g, w1, b1, w2, b2)


def _edge2_body(mji1, agg1, rbf, wjj, bjj, l2, wji2, bji2, l3,
                t2_o, mji2_o, r3_o):
    m2 = mji1[:] + agg1[:]
    x = rbf[:]
    t2_o[:] = _silu(_dot(m2, wjj[:]) + bjj[:]) * _dot(x, l2[:])
    mji2_o[:] = _silu(_dot(m2, wji2[:]) + bji2[:])
    r3_o[:] = _dot(x, l3[:])


def _tc_edge2(mji1, agg1, rbf, wjj, bjj, l2, wji2, bji2, l3):
    n, blk = rbf.shape[0], 3200
    return pl.pallas_call(
        _edge2_body,
        grid=(n // blk,),
        in_specs=[_row_spec(blk)] * 3
        + [_w_spec((DIM, DIM)), _w_spec((1, DIM)), _w_spec((DIM, DIM)),
           _w_spec((DIM, DIM)), _w_spec((1, DIM)), _w_spec((DIM, DIM))],
        out_specs=[_row_spec(blk)] * 3,
        out_shape=[jax.ShapeDtypeStruct((n, DIM), F32)] * 3,
    )(mji1, agg1, rbf, wjj, bjj, l2, wji2, bji2, l3)


def _edge3_body(mji2, agg2, r3, m3_o):
    m3_o[:] = r3[:] * (mji2[:] + agg2[:])


def _tc_edge3(mji2, agg2, r3):
    n, blk = mji2.shape[0], 4000
    return pl.pallas_call(
        _edge3_body,
        grid=(n // blk,),
        in_specs=[_row_spec(blk)] * 3,
        out_specs=_row_spec(blk),
        out_shape=jax.ShapeDtypeStruct((n, DIM), F32),
    )(mji2, agg2, r3)


def _node2_body(pa, pb, h,
                r1w1, r1b1, r1w2, r1b2,
                hw, hb,
                r2w1, r2b1, r2w2, r2b2,
                r3w1, r3b1, r3w2, r3b2,
                yw1, yb1, yw2, yb2, yw3, yb3,
                wy, by,
                h_o, y_o):
    def res(x, w1, b1, w2, b2):
        z = _silu(_dot(x, w1[:]) + b1[:])
        z = _silu(_dot(z, w2[:]) + b2[:])
        return z + x

    t = pa[:] + pb[:]
    t = res(t, r1w1, r1b1, r1w2, r1b2)
    t = _silu(_dot(t, hw[:]) + hb[:]) + h[:]
    t = res(t, r2w1, r2b1, r2w2, r2b2)
    t = res(t, r3w1, r3b1, r3w2, r3b2)
    h_o[:] = t
    z = _silu(_dot(t, yw1[:]) + yb1[:])
    z = _silu(_dot(z, yw2[:]) + yb2[:])
    z = _silu(_dot(z, yw3[:]) + yb3[:])
    y_o[:] = _dot(z, wy[:]) + by[:]


def _tc_node2(pa, pb, h, weights):
    n, blk = h.shape[0], 2000
    wspecs = []
    for w in weights:
        wspecs.append(_w_spec(w.shape))
    return pl.pallas_call(
        _node2_body,
        grid=(n // blk,),
        in_specs=[_row_spec(blk)] * 3 + wspecs,
        out_specs=[_row_spec(blk), pl.BlockSpec((blk, 1), lambda b: (b, 0))],
        out_shape=[jax.ShapeDtypeStruct((n, DIM), F32),
                   jax.ShapeDtypeStruct((n, 1), F32)],
    )(pa, pb, h, *weights)


_SC_MESH = dict(core_axis_name="c", subcore_axis_name="s",
                num_cores=NC, num_subcores=NS)
_WIN = 128


def _sc_gather(table, idx):
    b = idx.shape[0]
    per_w = b // NW
    n_full, tail = divmod(per_w, _WIN)
    mesh = plsc.VectorSubcoreMesh(**_SC_MESH)

    n_pairs = n_full // 2
    odd = n_full - 2 * n_pairs

    scratch = [
        [pltpu.VMEM((_WIN,), jnp.int32)] * 2,
        [pltpu.VMEM((_WIN, DIM), F32)] * 2,
        [pltpu.SemaphoreType.DMA] * 2,
    ]
    if tail:
        scratch += [pltpu.VMEM((tail,), jnp.int32), pltpu.VMEM((tail, DIM), F32)]

    @functools.partial(
        pl.kernel, mesh=mesh,
        out_type=jax.ShapeDtypeStruct((b, DIM), F32),
        scratch_types=scratch,
    )
    def k(table_hbm, idx_hbm, out_hbm, idx_v, rows_v, sem, *tail_bufs):
        wid = lax.axis_index("s") * NC + lax.axis_index("c")
        base = wid * per_w

        def gstart(off, s):
            pltpu.sync_copy(idx_hbm.at[pl.ds(off, _WIN)], idx_v[s])
            pltpu.async_copy(table_hbm.at[idx_v[s]], rows_v[s], sem[s])

        def gfinish(off, s):
            pltpu.make_async_copy(table_hbm.at[idx_v[s]], rows_v[s],
                                  sem[s]).wait()
            pltpu.sync_copy(rows_v[s], out_hbm.at[pl.ds(off, _WIN)])

        def win_sync(off, w, iv, rv):
            pltpu.sync_copy(idx_hbm.at[pl.ds(off, w)], iv)
            pltpu.async_copy(table_hbm.at[iv], rv, sem[0]).wait()
            pltpu.sync_copy(rv, out_hbm.at[pl.ds(off, w)])

        if n_pairs:
            gstart(base, 0)

            def body(kh, carry):
                w0 = base + 2 * kh * _WIN
                gstart(w0 + _WIN, 1)
                gfinish(w0, 0)

                @pl.when(kh < n_pairs - 1)
                def _pf():
                    gstart(w0 + 2 * _WIN, 0)

                gfinish(w0 + _WIN, 1)
                return carry

            lax.fori_loop(0, n_pairs, body, 0)
        if odd:
            win_sync(base + 2 * n_pairs * _WIN, _WIN, idx_v[0], rows_v[0])
        if tail:
            win_sync(base + n_full * _WIN, tail, tail_bufs[0], tail_bufs[1])

    return k(table, idx)


_EC_C = 10240
_EC_DUMMY = 256
_EC_ROWS = _EC_C + _EC_DUMMY
_EC_CHUNKS = [(b, min(_EC_C, N_EDGES - b)) for b in range(0, N_EDGES, _EC_C)]


def _sc_segsum_edges(v, dst, zeros):
    t = v.shape[0]
    per_sc = len(_EC_CHUNKS) // NC
    per_tile = t // NS
    n_full, tail = divmod(per_tile, _WIN)
    zrows_tile = _EC_ROWS // NS
    mesh = plsc.VectorSubcoreMesh(**_SC_MESH)

    n_half = n_full // 2
    assert n_full == 2 * n_half, "window count must be even for 2-deep ring"

    scratch = [
        [pltpu.VMEM((_WIN,), jnp.int32)] * 2,
        [pltpu.VMEM((_WIN,), jnp.int32)] * 2,
        [pltpu.VMEM((_WIN, DIM), F32)] * 2,
        [pltpu.SemaphoreType.DMA] * 2,
        [pltpu.SemaphoreType.DMA] * 2,
        pltpu.VMEM_SHARED((_EC_ROWS, DIM), F32),
    ]
    if tail:
        scratch += [pltpu.VMEM((tail,), jnp.int32),
                    pltpu.VMEM((tail,), jnp.int32),
                    pltpu.VMEM((tail, DIM), F32)]

    @functools.partial(
        pl.kernel, mesh=mesh,
        out_type=jax.ShapeDtypeStruct((N_EDGES, DIM), F32),
        scratch_types=scratch,
    )
    def k(v_hbm, dst_hbm, z_hbm, out_hbm, idx_v, loc_v, val_v, sem_i, sem_v,
          acc, *tail_bufs):
        cid = lax.axis_index("c")
        sid = lax.axis_index("s")
        tbase = sid * per_tile

        def start(off, s):
            pltpu.async_copy(dst_hbm.at[pl.ds(off, _WIN)], idx_v[s], sem_i[s])
            pltpu.async_copy(v_hbm.at[pl.ds(off, _WIN)], val_v[s], sem_v[s])

        def locs(nv, iv, lv, cbase):
            for kk in range(nv):
                dv = iv[pl.ds(kk * 16, 16)]
                loc = dv - cbase
                ok = (loc >= 0) & (loc < _EC_C)
                dummy = _EC_C + (dv & (_EC_DUMMY - 1))
                lv[pl.ds(kk * 16, 16)] = jnp.where(ok, loc, dummy)

        def finish(s, cbase):
            pltpu.make_async_copy(dst_hbm.at[pl.ds(0, _WIN)], idx_v[s],
                                  sem_i[s]).wait()
            pltpu.make_async_copy(v_hbm.at[pl.ds(0, _WIN)], val_v[s],
                                  sem_v[s]).wait()
            locs(_WIN // 16, idx_v[s], loc_v[s], cbase)
            pltpu.sync_copy(val_v[s], acc.at[loc_v[s]], add=True)

        def win_sync(off, w, nv, iv, lv, vv, cbase):
            pltpu.sync_copy(dst_hbm.at[pl.ds(off, w)], iv)
            pltpu.sync_copy(v_hbm.at[pl.ds(off, w)], vv)
            locs(nv, iv, lv, cbase)
            pltpu.sync_copy(vv, acc.at[lv], add=True)

        def writeout(cbase, csize):
            orows_tile = csize // NS
            pltpu.sync_copy(
                acc.at[pl.ds(sid * orows_tile, orows_tile)],
                out_hbm.at[pl.ds(cbase + sid * orows_tile, orows_tile)])

        for p in range(per_sc):
            chunk = cid * per_sc + p
            cbase = chunk * _EC_C
            pltpu.sync_copy(
                z_hbm.at[pl.ds(sid * zrows_tile, zrows_tile)],
                acc.at[pl.ds(sid * zrows_tile, zrows_tile)])
            plsc.subcore_barrier()

            start(tbase, 0)

            def body(kh, carry):
                w0 = tbase + 2 * kh * _WIN
                start(w0 + _WIN, 1)
                finish(0, cbase)

                @pl.when(kh < n_half - 1)
                def _pf():
                    start(w0 + 2 * _WIN, 0)

                finish(1, cbase)
                return carry

            lax.fori_loop(0, n_half, body, 0)
            if tail:
                win_sync(tbase + n_full * _WIN, tail, tail // 16,
                         tail_bufs[0], tail_bufs[1], tail_bufs[2], cbase)
            plsc.subcore_barrier()
            if p < per_sc - 1:
                writeout(cbase, _EC_C)
            else:
                last0 = _EC_CHUNKS[per_sc - 1]
                last1 = _EC_CHUNKS[NC * per_sc - 1]

                @pl.when(cid == 0)
                def _w0():
                    writeout(last0[0], last0[1])

                @pl.when(cid == 1)
                def _w1():
                    writeout(last1[0], last1[1])
            plsc.subcore_barrier()

    return k(v, dst, zeros)


def _sc_segsum_nodes(v, dst, zeros):
    e = v.shape[0]
    per_sc = e // NC
    per_tile = per_sc // NS
    n_full, tail = divmod(per_tile, _WIN)
    nrows = 10240
    zrows_tile = nrows // NS
    mesh = plsc.VectorSubcoreMesh(**_SC_MESH)

    n_pairs = n_full // 2
    odd = n_full - 2 * n_pairs

    scratch = [
        [pltpu.VMEM((_WIN,), jnp.int32)] * 2,
        [pltpu.VMEM((_WIN, DIM), F32)] * 2,
        [pltpu.SemaphoreType.DMA] * 2,
        [pltpu.SemaphoreType.DMA] * 2,
        pltpu.VMEM_SHARED((nrows, DIM), F32),
    ]
    if tail:
        scratch += [pltpu.VMEM((tail,), jnp.int32), pltpu.VMEM((tail, DIM), F32)]

    @functools.partial(
        pl.kernel, mesh=mesh,
        out_type=jax.ShapeDtypeStruct((NC * nrows, DIM), F32),
        scratch_types=scratch,
    )
    def k(v_hbm, dst_hbm, z_hbm, out_hbm, idx_v, val_v, sem_i, sem_v, acc,
          *tail_bufs):
        cid = lax.axis_index("c")
        sid = lax.axis_index("s")
        tbase = cid * per_sc + sid * per_tile

        pltpu.sync_copy(z_hbm.at[pl.ds(sid * zrows_tile, zrows_tile)],
                        acc.at[pl.ds(sid * zrows_tile, zrows_tile)])
        plsc.subcore_barrier()

        def start(off, s):
            pltpu.async_copy(dst_hbm.at[pl.ds(off, _WIN)], idx_v[s], sem_i[s])
            pltpu.async_copy(v_hbm.at[pl.ds(off, _WIN)], val_v[s], sem_v[s])

        def finish(s):
            pltpu.make_async_copy(dst_hbm.at[pl.ds(0, _WIN)], idx_v[s],
                                  sem_i[s]).wait()
            pltpu.make_async_copy(v_hbm.at[pl.ds(0, _WIN)], val_v[s],
                                  sem_v[s]).wait()
            pltpu.sync_copy(val_v[s], acc.at[idx_v[s]], add=True)

        def win_sync(off, w, iv, vv):
            pltpu.sync_copy(dst_hbm.at[pl.ds(off, w)], iv)
            pltpu.sync_copy(v_hbm.at[pl.ds(off, w)], vv)
            pltpu.sync_copy(vv, acc.at[iv], add=True)

        if n_pairs:
            start(tbase, 0)

            def body(kh, carry):
                w0 = tbase + 2 * kh * _WIN
                start(w0 + _WIN, 1)
                finish(0)

                @pl.when(kh < n_pairs - 1)
                def _pf():
                    start(w0 + 2 * _WIN, 0)

                finish(1)
                return carry

            lax.fori_loop(0, n_pairs, body, 0)
        if odd:
            win_sync(tbase + 2 * n_pairs * _WIN, _WIN, idx_v[0], val_v[0])
        if tail:
            win_sync(tbase + n_full * _WIN, tail, tail_bufs[0], tail_bufs[1])
        plsc.subcore_barrier()
        pltpu.sync_copy(
            acc.at[pl.ds(sid * zrows_tile, zrows_tile)],
            out_hbm.at[pl.ds(cid * nrows + sid * zrows_tile, zrows_tile)])

    return k(v, dst, zeros)


def kernel(h, rbf, sbf1, sbf2, idx_kj, idx_ji_1, idx_jj, idx_ji_2,
           edge_index, params):
    p = params
    i32 = jnp.int32
    j = edge_index[0].astype(i32)
    i = edge_index[1].astype(i32)
    idx_kj = idx_kj.astype(i32)
    idx_ji_1 = idx_ji_1.astype(i32)
    idx_jj = idx_jj.astype(i32)
    idx_ji_2 = idx_ji_2.astype(i32)

    def wb(layer):
        w, b = layer
        return w, b.reshape(1, DIM)

    wh, bh = wb(p['h_mlp'][0])
    wkj, bkj = wb(p['mlp_kj'][0])
    wj1, bj1 = wb(p['mlp_ji_1'][0])
    wjj, bjj = wb(p['mlp_jj'][0])
    wj2, bj2 = wb(p['mlp_ji_2'][0])
    s1w1, s1b1 = wb(p['mlp_sbf1'][0])
    s1w2, s1b2 = wb(p['mlp_sbf1'][1])
    s2w1, s2b1 = wb(p['mlp_sbf2'][0])
    s2w2, s2b2 = wb(p['mlp_sbf2'][1])

    zeros = jnp.zeros((_EC_ROWS, DIM), F32)

    hh = _tc_node1(h, wh, bh)
    hh_i = _sc_gather(hh, i)
    hh_j = _sc_gather(hh, j)

    t1, mji1 = _tc_edge1(
        hh_i, hh_j, rbf,
        wkj[:DIM], wkj[DIM:2 * DIM], wkj[2 * DIM:], bkj,
        wj1[:DIM], wj1[DIM:2 * DIM], wj1[2 * DIM:], bj1,
        p['lin_rbf1'])

    g1 = _sc_gather(t1, idx_kj)
    v1 = _tc_trip(sbf1, g1, s1w1, s1b1, s1w2, s1b2)
    agg1 = _sc_segsum_edges(v1, idx_ji_1, zeros)

    t2, mji2, r3 = _tc_edge2(mji1, agg1, rbf, wjj, bjj, p['lin_rbf2'],
                             wj2, bj2, p['lin_rbf_out'])

    g2 = _sc_gather(t2, idx_jj)
    v2 = _tc_trip(sbf2, g2, s2w1, s2b1, s2w2, s2b2)
    agg2 = _sc_segsum_edges(v2, idx_ji_2, zeros)

    m3 = _tc_edge3(mji2, agg2, r3)
    hparts = _sc_segsum_nodes(m3, i, zeros)
    pa = hparts[:N_NODES]
    pb = hparts[10240:10240 + N_NODES]

    weights = []
    for (w1, b1), (w2, b2) in [(p['res1'][0], p['res1'][1])]:
        weights += [w1, b1.reshape(1, DIM), w2, b2.reshape(1, DIM)]
    weights += [wh, bh]
    for key in ('res2', 'res3'):
        (w1, b1), (w2, b2) = p[key]
        weights += [w1, b1.reshape(1, DIM), w2, b2.reshape(1, DIM)]
    for w, b in p['y_mlp']:
        weights += [w, b.reshape(1, DIM)]
    wy, by = p['y_W']
    weights += [wy, by.reshape(1, 1)]

    h_out, y = _tc_node2(pa, pb, h, weights)
    return (h_out, y)


# speedup vs baseline: 1.6725x; 1.0162x over previous
import functools

import jax
import jax.numpy as jnp
from jax import lax
from jax.experimental import pallas as pl
from jax.experimental.pallas import tpu as pltpu
from jax.experimental.pallas import tpu_sc as plsc

DIM = 128
N_NODES = 10000
N_EDGES = 160000
N_TRIPLETS = 320000

NC = 2
NS = 16
NW = NC * NS

F32 = jnp.float32


def _silu(x):
    return x * jax.nn.sigmoid(x)


def _dot(a, b):
    return jnp.dot(a, b, preferred_element_type=F32)


def _row_spec(blk):
    return pl.BlockSpec((blk, DIM), lambda b: (b, 0))


def _w_spec(shape):
    return pl.BlockSpec(shape, lambda b: tuple(0 for _ in shape))


def _node1_body(h_ref, w_ref, b_ref, o_ref):
    o_ref[:] = _silu(_dot(h_ref[:], w_ref[:]) + b_ref[:])


def _tc_node1(h, w, b):
    n, blk = h.shape[0], 2000
    return pl.pallas_call(
        _node1_body,
        grid=(n // blk,),
        in_specs=[_row_spec(blk), _w_spec((DIM, DIM)), _w_spec((1, DIM))],
        out_specs=_row_spec(blk),
        out_shape=jax.ShapeDtypeStruct((n, DIM), F32),
    )(h, w, b)


def _edge1_body(hi, hj, rbf, wa1, wb1, wc1, b1, wa2, wb2, wc2, b2, l1,
                t1_o, mji1_o):
    x = rbf[:]
    mkj = _silu(_dot(hi[:], wa1[:]) + _dot(hj[:], wb1[:]) + _dot(x, wc1[:])
                + b1[:])
    t1_o[:] = mkj * _dot(x, l1[:])
    mji1_o[:] = _silu(_dot(hi[:], wa2[:]) + _dot(hj[:], wb2[:])
                      + _dot(x, wc2[:]) + b2[:])


def _tc_edge1(hi, hj, rbf, wa1, wb1, wc1, b1, wa2, wb2, wc2, b2, l1):
    n, blk = rbf.shape[0], 6400
    ws = [_w_spec((DIM, DIM))] * 3 + [_w_spec((1, DIM))]
    return pl.pallas_call(
        _edge1_body,
        grid=(n // blk,),
        in_specs=[_row_spec(blk)] * 3 + ws + ws + [_w_spec((DIM, DIM))],
        out_specs=[_row_spec(blk)] * 2,
        out_shape=[jax.ShapeDtypeStruct((n, DIM), F32)] * 2,
    )(hi, hj, rbf, wa1, wb1, wc1, b1, wa2, wb2, wc2, b2, l1)


def _trip_body(sbf, g, w1, b1, w2, b2, v_o):
    x = _silu(_dot(sbf[:], w1[:]) + b1[:])
    x = _silu(_dot(x, w2[:]) + b2[:])
    v_o[:] = x * g[:]


def _tc_trip(sbf, g, w1, b1, w2, b2):
    n, blk = sbf.shape[0], 6400
    return pl.pallas_call(
        _trip_body,
        grid=(n // blk,),
        in_specs=[_row_spec(blk)] * 2 + [_w_spec((DIM, DIM)), _w_spec((1, DIM)),
                                         _w_spec((DIM, DIM)), _w_spec((1, DIM))],
        out_specs=_row_spec(blk),
        out_shape=jax.ShapeDtypeStruct((n, DIM), F32),
    )(sbf, g, w1, b1, w2, b2)


def _edge2_body(mji1, agg1, rbf, wjj, bjj, l2, wji2, bji2, l3,
                t2_o, mji2_o, r3_o):
    m2 = mji1[:] + agg1[:]
    x = rbf[:]
    t2_o[:] = _silu(_dot(m2, wjj[:]) + bjj[:]) * _dot(x, l2[:])
    mji2_o[:] = _silu(_dot(m2, wji2[:]) + bji2[:])
    r3_o[:] = _dot(x, l3[:])


def _tc_edge2(mji1, agg1, rbf, wjj, bjj, l2, wji2, bji2, l3):
    n, blk = rbf.shape[0], 6400
    return pl.pallas_call(
        _edge2_body,
        grid=(n // blk,),
        in_specs=[_row_spec(blk)] * 3
        + [_w_spec((DIM, DIM)), _w_spec((1, DIM)), _w_spec((DIM, DIM)),
           _w_spec((DIM, DIM)), _w_spec((1, DIM)), _w_spec((DIM, DIM))],
        out_specs=[_row_spec(blk)] * 3,
        out_shape=[jax.ShapeDtypeStruct((n, DIM), F32)] * 3,
    )(mji1, agg1, rbf, wjj, bjj, l2, wji2, bji2, l3)


def _edge3_body(mji2, agg2, r3, m3_o):
    m3_o[:] = r3[:] * (mji2[:] + agg2[:])


def _tc_edge3(mji2, agg2, r3):
    n, blk = mji2.shape[0], 8000
    return pl.pallas_call(
        _edge3_body,
        grid=(n // blk,),
        in_specs=[_row_spec(blk)] * 3,
        out_specs=_row_spec(blk),
        out_shape=jax.ShapeDtypeStruct((n, DIM), F32),
    )(mji2, agg2, r3)


def _node2_body(pa, pb, h,
                r1w1, r1b1, r1w2, r1b2,
                hw, hb,
                r2w1, r2b1, r2w2, r2b2,
                r3w1, r3b1, r3w2, r3b2,
                yw1, yb1, yw2, yb2, yw3, yb3,
                wy, by,
                h_o, y_o):
    def res(x, w1, b1, w2, b2):
        z = _silu(_dot(x, w1[:]) + b1[:])
        z = _silu(_dot(z, w2[:]) + b2[:])
        return z + x

    t = pa[:] + pb[:]
    t = res(t, r1w1, r1b1, r1w2, r1b2)
    t = _silu(_dot(t, hw[:]) + hb[:]) + h[:]
    t = res(t, r2w1, r2b1, r2w2, r2b2)
    t = res(t, r3w1, r3b1, r3w2, r3b2)
    h_o[:] = t
    z = _silu(_dot(t, yw1[:]) + yb1[:])
    z = _silu(_dot(z, yw2[:]) + yb2[:])
    z = _silu(_dot(z, yw3[:]) + yb3[:])
    y_o[:] = _dot(z, wy[:]) + by[:]


def _tc_node2(pa, pb, h, weights):
    n, blk = h.shape[0], 2000
    wspecs = []
    for w in weights:
        wspecs.append(_w_spec(w.shape))
    return pl.pallas_call(
        _node2_body,
        grid=(n // blk,),
        in_specs=[_row_spec(blk)] * 3 + wspecs,
        out_specs=[_row_spec(blk), pl.BlockSpec((blk, 1), lambda b: (b, 0))],
        out_shape=[jax.ShapeDtypeStruct((n, DIM), F32),
                   jax.ShapeDtypeStruct((n, 1), F32)],
    )(pa, pb, h, *weights)


_SC_MESH = dict(core_axis_name="c", subcore_axis_name="s",
                num_cores=NC, num_subcores=NS)
_WIN = 128


def _sc_gather(table, idx):
    b = idx.shape[0]
    per_w = b // NW
    n_full, tail = divmod(per_w, _WIN)
    mesh = plsc.VectorSubcoreMesh(**_SC_MESH)

    n_pairs = n_full // 2
    odd = n_full - 2 * n_pairs

    scratch = [
        [pltpu.VMEM((_WIN,), jnp.int32)] * 2,
        [pltpu.VMEM((_WIN, DIM), F32)] * 2,
        [pltpu.SemaphoreType.DMA] * 2,
    ]
    if tail:
        scratch += [pltpu.VMEM((tail,), jnp.int32), pltpu.VMEM((tail, DIM), F32)]

    @functools.partial(
        pl.kernel, mesh=mesh,
        out_type=jax.ShapeDtypeStruct((b, DIM), F32),
        scratch_types=scratch,
    )
    def k(table_hbm, idx_hbm, out_hbm, idx_v, rows_v, sem, *tail_bufs):
        wid = lax.axis_index("s") * NC + lax.axis_index("c")
        base = wid * per_w

        def gstart(off, s):
            pltpu.sync_copy(idx_hbm.at[pl.ds(off, _WIN)], idx_v[s])
            pltpu.async_copy(table_hbm.at[idx_v[s]], rows_v[s], sem[s])

        def gfinish(off, s):
            pltpu.make_async_copy(table_hbm.at[idx_v[s]], rows_v[s],
                                  sem[s]).wait()
            pltpu.sync_copy(rows_v[s], out_hbm.at[pl.ds(off, _WIN)])

        def win_sync(off, w, iv, rv):
            pltpu.sync_copy(idx_hbm.at[pl.ds(off, w)], iv)
            pltpu.async_copy(table_hbm.at[iv], rv, sem[0]).wait()
            pltpu.sync_copy(rv, out_hbm.at[pl.ds(off, w)])

        if n_pairs:
            gstart(base, 0)

            def body(kh, carry):
                w0 = base + 2 * kh * _WIN
                gstart(w0 + _WIN, 1)
                gfinish(w0, 0)

                @pl.when(kh < n_pairs - 1)
                def _pf():
                    gstart(w0 + 2 * _WIN, 0)

                gfinish(w0 + _WIN, 1)
                return carry

            lax.fori_loop(0, n_pairs, body, 0)
        if odd:
            win_sync(base + 2 * n_pairs * _WIN, _WIN, idx_v[0], rows_v[0])
        if tail:
            win_sync(base + n_full * _WIN, tail, tail_bufs[0], tail_bufs[1])

    return k(table, idx)


_EC_C = 10240
_EC_DUMMY = 256
_EC_ROWS = _EC_C + _EC_DUMMY
_EC_CHUNKS = [(b, min(_EC_C, N_EDGES - b)) for b in range(0, N_EDGES, _EC_C)]


def _sc_segsum_edges(v, dst, zeros):
    t = v.shape[0]
    per_sc = len(_EC_CHUNKS) // NC
    per_tile = t // NS
    n_full, tail = divmod(per_tile, _WIN)
    zrows_tile = _EC_ROWS // NS
    mesh = plsc.VectorSubcoreMesh(**_SC_MESH)

    n_half = n_full // 2
    assert n_full == 2 * n_half, "window count must be even for 2-deep ring"

    scratch = [
        [pltpu.VMEM((_WIN,), jnp.int32)] * 2,
        [pltpu.VMEM((_WIN,), jnp.int32)] * 2,
        [pltpu.VMEM((_WIN, DIM), F32)] * 2,
        [pltpu.SemaphoreType.DMA] * 2,
        [pltpu.SemaphoreType.DMA] * 2,
        pltpu.VMEM_SHARED((_EC_ROWS, DIM), F32),
    ]
    if tail:
        scratch += [pltpu.VMEM((tail,), jnp.int32),
                    pltpu.VMEM((tail,), jnp.int32),
                    pltpu.VMEM((tail, DIM), F32)]

    @functools.partial(
        pl.kernel, mesh=mesh,
        out_type=jax.ShapeDtypeStruct((N_EDGES, DIM), F32),
        scratch_types=scratch,
    )
    def k(v_hbm, dst_hbm, z_hbm, out_hbm, idx_v, loc_v, val_v, sem_i, sem_v,
          acc, *tail_bufs):
        cid = lax.axis_index("c")
        sid = lax.axis_index("s")
        tbase = sid * per_tile

        def start(off, s):
            pltpu.async_copy(dst_hbm.at[pl.ds(off, _WIN)], idx_v[s], sem_i[s])
            pltpu.async_copy(v_hbm.at[pl.ds(off, _WIN)], val_v[s], sem_v[s])

        def locs(nv, iv, lv, cbase):
            for kk in range(nv):
                dv = iv[pl.ds(kk * 16, 16)]
                loc = dv - cbase
                ok = (loc >= 0) & (loc < _EC_C)
                dummy = _EC_C + (dv & (_EC_DUMMY - 1))
                lv[pl.ds(kk * 16, 16)] = jnp.where(ok, loc, dummy)

        def finish(s, cbase):
            pltpu.make_async_copy(dst_hbm.at[pl.ds(0, _WIN)], idx_v[s],
                                  sem_i[s]).wait()
            pltpu.make_async_copy(v_hbm.at[pl.ds(0, _WIN)], val_v[s],
                                  sem_v[s]).wait()
            locs(_WIN // 16, idx_v[s], loc_v[s], cbase)
            pltpu.sync_copy(val_v[s], acc.at[loc_v[s]], add=True)

        def win_sync(off, w, nv, iv, lv, vv, cbase):
            pltpu.sync_copy(dst_hbm.at[pl.ds(off, w)], iv)
            pltpu.sync_copy(v_hbm.at[pl.ds(off, w)], vv)
            locs(nv, iv, lv, cbase)
            pltpu.sync_copy(vv, acc.at[lv], add=True)

        def writeout(cbase, csize):
            orows_tile = csize // NS
            pltpu.sync_copy(
                acc.at[pl.ds(sid * orows_tile, orows_tile)],
                out_hbm.at[pl.ds(cbase + sid * orows_tile, orows_tile)])

        for p in range(per_sc):
            chunk = cid * per_sc + p
            cbase = chunk * _EC_C
            pltpu.sync_copy(
                z_hbm.at[pl.ds(sid * zrows_tile, zrows_tile)],
                acc.at[pl.ds(sid * zrows_tile, zrows_tile)])
            plsc.subcore_barrier()

            start(tbase, 0)

            def body(kh, carry):
                w0 = tbase + 2 * kh * _WIN
                start(w0 + _WIN, 1)
                finish(0, cbase)

                @pl.when(kh < n_half - 1)
                def _pf():
                    start(w0 + 2 * _WIN, 0)

                finish(1, cbase)
                return carry

            lax.fori_loop(0, n_half, body, 0)
            if tail:
                win_sync(tbase + n_full * _WIN, tail, tail // 16,
                         tail_bufs[0], tail_bufs[1], tail_bufs[2], cbase)
            plsc.subcore_barrier()
            if p < per_sc - 1:
                writeout(cbase, _EC_C)
            else:
                last0 = _EC_CHUNKS[per_sc - 1]
                last1 = _EC_CHUNKS[NC * per_sc - 1]

                @pl.when(cid == 0)
                def _w0():
                    writeout(last0[0], last0[1])

                @pl.when(cid == 1)
                def _w1():
                    writeout(last1[0], last1[1])
            plsc.subcore_barrier()

    return k(v, dst, zeros)


def _sc_segsum_nodes(v, dst, zeros):
    e = v.shape[0]
    per_sc = e // NC
    per_tile = per_sc // NS
    n_full, tail = divmod(per_tile, _WIN)
    nrows = 10240
    zrows_tile = nrows // NS
    mesh = plsc.VectorSubcoreMesh(**_SC_MESH)

    n_pairs = n_full // 2
    odd = n_full - 2 * n_pairs

    scratch = [
        [pltpu.VMEM((_WIN,), jnp.int32)] * 2,
        [pltpu.VMEM((_WIN, DIM), F32)] * 2,
        [pltpu.SemaphoreType.DMA] * 2,
        [pltpu.SemaphoreType.DMA] * 2,
        pltpu.VMEM_SHARED((nrows, DIM), F32),
    ]
    if tail:
        scratch += [pltpu.VMEM((tail,), jnp.int32), pltpu.VMEM((tail, DIM), F32)]

    @functools.partial(
        pl.kernel, mesh=mesh,
        out_type=jax.ShapeDtypeStruct((NC * nrows, DIM), F32),
        scratch_types=scratch,
    )
    def k(v_hbm, dst_hbm, z_hbm, out_hbm, idx_v, val_v, sem_i, sem_v, acc,
          *tail_bufs):
        cid = lax.axis_index("c")
        sid = lax.axis_index("s")
        tbase = cid * per_sc + sid * per_tile

        pltpu.sync_copy(z_hbm.at[pl.ds(sid * zrows_tile, zrows_tile)],
                        acc.at[pl.ds(sid * zrows_tile, zrows_tile)])
        plsc.subcore_barrier()

        def start(off, s):
            pltpu.async_copy(dst_hbm.at[pl.ds(off, _WIN)], idx_v[s], sem_i[s])
            pltpu.async_copy(v_hbm.at[pl.ds(off, _WIN)], val_v[s], sem_v[s])

        def finish(s):
            pltpu.make_async_copy(dst_hbm.at[pl.ds(0, _WIN)], idx_v[s],
                                  sem_i[s]).wait()
            pltpu.make_async_copy(v_hbm.at[pl.ds(0, _WIN)], val_v[s],
                                  sem_v[s]).wait()
            pltpu.sync_copy(val_v[s], acc.at[idx_v[s]], add=True)

        def win_sync(off, w, iv, vv):
            pltpu.sync_copy(dst_hbm.at[pl.ds(off, w)], iv)
            pltpu.sync_copy(v_hbm.at[pl.ds(off, w)], vv)
            pltpu.sync_copy(vv, acc.at[iv], add=True)

        if n_pairs:
            start(tbase, 0)

            def body(kh, carry):
                w0 = tbase + 2 * kh * _WIN
                start(w0 + _WIN, 1)
                finish(0)

                @pl.when(kh < n_pairs - 1)
                def _pf():
                    start(w0 + 2 * _WIN, 0)

                finish(1)
                return carry

            lax.fori_loop(0, n_pairs, body, 0)
        if odd:
            win_sync(tbase + 2 * n_pairs * _WIN, _WIN, idx_v[0], val_v[0])
        if tail:
            win_sync(tbase + n_full * _WIN, tail, tail_bufs[0], tail_bufs[1])
        plsc.subcore_barrier()
        pltpu.sync_copy(
            acc.at[pl.ds(sid * zrows_tile, zrows_tile)],
            out_hbm.at[pl.ds(cid * nrows + sid * zrows_tile, zrows_tile)])

    return k(v, dst, zeros)


def kernel(h, rbf, sbf1, sbf2, idx_kj, idx_ji_1, idx_jj, idx_ji_2,
           edge_index, params):
    p = params
    i32 = jnp.int32
    j = edge_index[0].astype(i32)
    i = edge_index[1].astype(i32)
    idx_kj = idx_kj.astype(i32)
    idx_ji_1 = idx_ji_1.astype(i32)
    idx_jj = idx_jj.astype(i32)
    idx_ji_2 = idx_ji_2.astype(i32)

    def wb(layer):
        w, b = layer
        return w, b.reshape(1, DIM)

    wh, bh = wb(p['h_mlp'][0])
    wkj, bkj = wb(p['mlp_kj'][0])
    wj1, bj1 = wb(p['mlp_ji_1'][0])
    wjj, bjj = wb(p['mlp_jj'][0])
    wj2, bj2 = wb(p['mlp_ji_2'][0])
    s1w1, s1b1 = wb(p['mlp_sbf1'][0])
    s1w2, s1b2 = wb(p['mlp_sbf1'][1])
    s2w1, s2b1 = wb(p['mlp_sbf2'][0])
    s2w2, s2b2 = wb(p['mlp_sbf2'][1])

    zeros = jnp.zeros((_EC_ROWS, DIM), F32)

    hh = _tc_node1(h, wh, bh)
    hh_i = _sc_gather(hh, i)
    hh_j = _sc_gather(hh, j)

    t1, mji1 = _tc_edge1(
        hh_i, hh_j, rbf,
        wkj[:DIM], wkj[DIM:2 * DIM], wkj[2 * DIM:], bkj,
        wj1[:DIM], wj1[DIM:2 * DIM], wj1[2 * DIM:], bj1,
        p['lin_rbf1'])

    g1 = _sc_gather(t1, idx_kj)
    v1 = _tc_trip(sbf1, g1, s1w1, s1b1, s1w2, s1b2)
    agg1 = _sc_segsum_edges(v1, idx_ji_1, zeros)

    t2, mji2, r3 = _tc_edge2(mji1, agg1, rbf, wjj, bjj, p['lin_rbf2'],
                             wj2, bj2, p['lin_rbf_out'])

    g2 = _sc_gather(t2, idx_jj)
    v2 = _tc_trip(sbf2, g2, s2w1, s2b1, s2w2, s2b2)
    agg2 = _sc_segsum_edges(v2, idx_ji_2, zeros)

    m3 = _tc_edge3(mji2, agg2, r3)
    hparts = _sc_segsum_nodes(m3, i, zeros)
    pa = hparts[:N_NODES]
    pb = hparts[10240:10240 + N_NODES]

    weights = []
    for (w1, b1), (w2, b2) in [(p['res1'][0], p['res1'][1])]:
        weights += [w1, b1.reshape(1, DIM), w2, b2.reshape(1, DIM)]
    weights += [wh, bh]
    for key in ('res2', 'res3'):
        (w1, b1), (w2, b2) = p[key]
        weights += [w1, b1.reshape(1, DIM), w2, b2.reshape(1, DIM)]
    for w, b in p['y_mlp']:
        weights += [w, b.reshape(1, DIM)]
    wy, by = p['y_W']
    weights += [wy, by.reshape(1, 1)]

    h_out, y = _tc_node2(pa, pb, h, weights)
    return (h_out, y)
